# Initial kernel scaffold; baseline (speedup 1.0000x reference)
import functools

import jax
import jax.numpy as jnp
from jax import lax
from jax.experimental import pallas as pl
from jax.experimental.pallas import tpu as pltpu
from jax.experimental.pallas import tpu_sc as plsc

N = 100000
NE = 3200000
NP = 100352
TS = NP // 16
EROWS = 25088
NEP = EROWS * 128
KB = 16
ROWS_PER_TILE = EROWS // 16
ROWS_PER_TILE_HALF = EROWS // 32

_SC_MESH = plsc.VectorSubcoreMesh(core_axis_name="c", subcore_axis_name="s")


def _rsqrt16(d):
    i = plsc.bitcast(d, jnp.int32)
    i = jnp.int32(0x5F3759DF) - lax.shift_right_logical(i, jnp.int32(1))
    y = plsc.bitcast(i, jnp.float32)
    for _ in range(3):
        y = y * (1.5 - 0.5 * d * y * y)
    return y


@functools.partial(
    pl.kernel,
    out_type=(
        jax.ShapeDtypeStruct((NP,), jnp.float32),
        jax.ShapeDtypeStruct((NP,), jnp.float32),
    ),
    mesh=_SC_MESH,
    scratch_types=[
        pltpu.VMEM((KB, 128), jnp.int32),
        pltpu.VMEM((128,), jnp.float32),
        pltpu.VMEM((TS,), jnp.float32),
        pltpu.VMEM_SHARED((NP,), jnp.float32),
        pltpu.SemaphoreType.DMA,
    ],
)
def _deg_kernel(dst_hbm, deg_a, deg_b, dbuf, ones, slbuf, acc, sem):
    c = lax.axis_index("c")
    s = lax.axis_index("s")
    off = pl.multiple_of(s * TS, 8)

    def fill(i, _):
        ix = pl.ds(pl.multiple_of(i * 16, 16), 16)
        slbuf[ix] = jnp.zeros((16,), jnp.float32)
        return 0

    lax.fori_loop(0, TS // 16, fill, 0)
    for j in range(8):
        ones[pl.ds(j * 16, 16)] = jnp.ones((16,), jnp.float32)
    pltpu.sync_copy(slbuf, acc.at[pl.ds(off, TS)])
    plsc.subcore_barrier()

    base = c * (16 * ROWS_PER_TILE_HALF) + s * ROWS_PER_TILE_HALF

    def blk(b, _):
        row0 = base + b * KB
        pltpu.sync_copy(dst_hbm.at[pl.ds(row0, KB)], dbuf)
        descs = [
            pltpu.async_copy(ones, acc.at[dbuf.at[j]], sem, add=True)
            for j in range(KB)
        ]
        for d in descs:
            d.wait()
        return 0

    lax.fori_loop(0, ROWS_PER_TILE_HALF // KB, blk, 0)
    plsc.subcore_barrier()

    @pl.when(c == 0)
    def _():
        pltpu.sync_copy(acc.at[pl.ds(off, TS)], deg_a.at[pl.ds(off, TS)])

    @pl.when(c == 1)
    def _():
        pltpu.sync_copy(acc.at[pl.ds(off, TS)], deg_b.at[pl.ds(off, TS)])


@functools.partial(
    pl.kernel,
    out_type=(
        jax.ShapeDtypeStruct((NP,), jnp.float32),
        jax.ShapeDtypeStruct((NP,), jnp.float32),
        jax.ShapeDtypeStruct((NP,), jnp.float32),
    ),
    mesh=_SC_MESH,
    scratch_types=[
        pltpu.VMEM((KB, 128), jnp.int32),
        pltpu.VMEM((KB, 128), jnp.int32),
        pltpu.VMEM((KB, 128), jnp.float32),
        pltpu.VMEM((TS,), jnp.float32),
        pltpu.VMEM((TS,), jnp.float32),
        pltpu.VMEM((TS,), jnp.float32),
        pltpu.VMEM_SHARED((NP,), jnp.float32),
        pltpu.VMEM_SHARED((NP,), jnp.float32),
        pltpu.VMEM_SHARED((NP,), jnp.float32),
        pltpu.VMEM_SHARED((NP,), jnp.float32),
        pltpu.SemaphoreType.DMA,
    ],
)
def _chain_kernel(src_hbm, dst_hbm, h0_hbm, deg_a, deg_b, s1o, s2o, v3o,
                  srcb, dstb, vals, t_a, t_b, t_c, w_sp, acc_sp, q_sp,
                  y_sp, sem):
    c = lax.axis_index("c")
    s = lax.axis_index("s")
    off = pl.multiple_of(s * TS, 8)
    sl = pl.ds(off, TS)

    pltpu.sync_copy(deg_a.at[sl], t_a)
    pltpu.sync_copy(deg_b.at[sl], t_b)

    @pl.when(c == 1)
    def _():
        pltpu.sync_copy(h0_hbm.at[sl], t_c)

    def ew0(i, _):
        ix = pl.ds(pl.multiple_of(i * 16, 16), 16)
        d = t_a[ix] + t_b[ix] + 1.0
        y = _rsqrt16(d)
        t_a[ix] = y
        t_b[ix] = y * y
        return 0

    lax.fori_loop(0, TS // 16, ew0, 0)
    pltpu.sync_copy(t_a, y_sp.at[sl])
    pltpu.sync_copy(t_b, q_sp.at[sl])

    @pl.when(c == 0)
    def _():
        pltpu.sync_copy(t_a, w_sp.at[sl])
        pltpu.sync_copy(t_a, acc_sp.at[sl])

    @pl.when(c == 1)
    def _():
        def mul(i, _):
            ix = pl.ds(pl.multiple_of(i * 16, 16), 16)
            t_c[ix] = t_c[ix] * t_a[ix]
            return 0

        lax.fori_loop(0, TS // 16, mul, 0)
        pltpu.sync_copy(t_c, w_sp.at[sl])
        pltpu.sync_copy(t_c, acc_sp.at[sl])

    plsc.subcore_barrier()

    ebase = s * ROWS_PER_TILE

    def edge_pass():
        def blk(b, _):
            row0 = ebase + b * KB
            pltpu.sync_copy(src_hbm.at[pl.ds(row0, KB)], srcb)
            pltpu.sync_copy(dst_hbm.at[pl.ds(row0, KB)], dstb)
            descs = [
                pltpu.async_copy(w_sp.at[srcb.at[j]], vals.at[j], sem)
                for j in range(KB)
            ]
            for d in descs:
                d.wait()
            descs = [
                pltpu.async_copy(vals.at[j], acc_sp.at[dstb.at[j]], sem,
                                 add=True)
                for j in range(KB)
            ]
            for d in descs:
                d.wait()
            return 0

        lax.fori_loop(0, ROWS_PER_TILE // KB, blk, 0)

    def ew(out_ref, cont):
        pltpu.sync_copy(acc_sp.at[sl], t_a)
        pltpu.sync_copy(q_sp.at[sl], t_b)
        if out_ref is not None:
            pltpu.sync_copy(y_sp.at[sl], t_c)

        def body(i, _):
            ix = pl.ds(pl.multiple_of(i * 16, 16), 16)
            a = t_a[ix]
            if out_ref is not None:
                t_c[ix] = a * t_c[ix]
            if cont:
                t_a[ix] = a * t_b[ix]
            return 0

        lax.fori_loop(0, TS // 16, body, 0)
        if out_ref is not None:
            pltpu.sync_copy(t_c, out_ref.at[sl])
        if cont:
            pltpu.sync_copy(t_a, w_sp.at[sl])
            pltpu.sync_copy(t_a, acc_sp.at[sl])

    @pl.when(c == 0)
    def _():
        edge_pass()
        plsc.subcore_barrier()
        ew(s1o, True)
        plsc.subcore_barrier()
        edge_pass()
        plsc.subcore_barrier()
        ew(s2o, False)
        plsc.subcore_barrier()
        plsc.subcore_barrier()

    @pl.when(c == 1)
    def _():
        edge_pass()
        plsc.subcore_barrier()
        ew(None, True)
        plsc.subcore_barrier()
        edge_pass()
        plsc.subcore_barrier()
        ew(None, True)
        plsc.subcore_barrier()
        edge_pass()
        plsc.subcore_barrier()
        ew(v3o, False)


def _enc_body(x_ref, wi_ref, bi_ref, w2_ref, w3_ref, u1_ref, u2_ref,
              u3_ref, wl1_ref, blr_ref, h0_ref, p4_ref):
    f32 = jnp.float32
    h = jnp.dot(x_ref[...], wi_ref[...], preferred_element_type=f32)
    h0_ref[...] = jax.nn.leaky_relu(h + bi_ref[...])
    t = jnp.dot(u1_ref[...], w2_ref[...], preferred_element_type=f32)
    g = (jnp.dot(t, w3_ref[...], preferred_element_type=f32)
         + jnp.dot(u2_ref[...], w3_ref[...], preferred_element_type=f32)
         + u3_ref[...])
    p4_ref[...] = (jnp.dot(g, wl1_ref[...], preferred_element_type=f32)
                   + blr_ref[...])


def _enc_call(x, wi, bi, w2p, w3p, u1, u2, u3, wl1p, blr):
    return pl.pallas_call(
        _enc_body,
        out_shape=(
            jax.ShapeDtypeStruct((1000, 100), jnp.float32),
            jax.ShapeDtypeStruct((8, 512), jnp.float32),
        ),
    )(x, wi, bi, w2p, w3p, u1, u2, u3, wl1p, blr)


def _dec_body(s4_ref, p4_ref, wl2_ref, bl2_ref, wl3_ref, bl3_ref, out_ref):
    f32 = jnp.float32
    z = jnp.dot(s4_ref[...], p4_ref[...], preferred_element_type=f32)
    z = jax.nn.leaky_relu(z)
    g = jnp.dot(z, wl2_ref[...], preferred_element_type=f32) + bl2_ref[...]
    g = jax.nn.leaky_relu(g)
    o = jnp.dot(g, wl3_ref[...], preferred_element_type=f32) + bl3_ref[...]
    out_ref[...] = jax.nn.leaky_relu(o)


def _dec_call(s4, p4, wl2, bl2, wl3, bl3):
    rows = 2000
    grid = (N // rows,)
    return pl.pallas_call(
        _dec_body,
        grid=grid,
        in_specs=[
            pl.BlockSpec((rows, 8), lambda i: (i, 0)),
            pl.BlockSpec((8, 512), lambda i: (0, 0)),
            pl.BlockSpec((512, 128), lambda i: (0, 0)),
            pl.BlockSpec((1, 128), lambda i: (0, 0)),
            pl.BlockSpec((128, 512), lambda i: (0, 0)),
            pl.BlockSpec((1, 512), lambda i: (0, 0)),
        ],
        out_specs=pl.BlockSpec((rows, 512), lambda i: (i, 0)),
        out_shape=jax.ShapeDtypeStruct((N, 512), jnp.float32),
    )(s4, p4, wl2, bl2, wl3, bl3)


def kernel(x, edge_index, W_inv, b_inv, W1, b1, W2, b2, W3, b3, Wl1, bl1,
           Wl2, bl2, Wl3, bl3):
    f32 = jnp.float32

    pad_idx = N + (jnp.arange(NEP - NE, dtype=jnp.int32) % (NP - N))
    srcp = jnp.concatenate([edge_index[0], pad_idx]).reshape(EROWS, 128)
    dstp = jnp.concatenate([edge_index[1], pad_idx]).reshape(EROWS, 128)

    w2p = jnp.zeros((16, 16), f32).at[:9, :3].set(W2)
    w3p = jnp.zeros((16, 16), f32).at[:3, :3].set(W3)
    u1 = jnp.zeros((8, 16), f32).at[0, :9].set(W1[0]).at[1, :9].set(b1)
    u2 = jnp.zeros((8, 16), f32).at[2, :3].set(b2)
    u3 = jnp.zeros((8, 16), f32).at[3, :3].set(b3)
    wl1p = jnp.zeros((16, 512), f32).at[:3].set(Wl1)
    blr = jnp.zeros((8, 512), f32).at[3].set(bl1)

    h0m, p4 = _enc_call(x, W_inv, b_inv.reshape(1, 100), w2p, w3p, u1, u2,
                        u3, wl1p, blr)
    h0p = jnp.concatenate([h0m.reshape(-1), jnp.zeros((NP - N,), f32)])

    deg_a, deg_b = _deg_kernel(dstp)
    s1, s2, v3 = _chain_kernel(srcp, dstp, h0p, deg_a, deg_b)

    s4 = jnp.stack(
        [v3[:N], s2[:N], s1[:N], jnp.ones((N,), f32)], axis=1)
    s4 = jnp.concatenate([s4, jnp.zeros((N, 4), f32)], axis=1)

    out = _dec_call(s4, p4, Wl2, bl2.reshape(1, 128), Wl3,
                    bl3.reshape(1, 512))
    return out, edge_index


# speedup vs baseline: 83.1610x; 83.1610x over previous
import functools

import jax
import jax.numpy as jnp
from jax import lax
from jax.experimental import pallas as pl
from jax.experimental.pallas import tpu as pltpu
from jax.experimental.pallas import tpu_sc as plsc

N = 100000
NE = 3200000
NP = 100352
TS = NP // 16
EROWS = 25088
NEP = EROWS * 128
KB = 16
ROWS_PER_TILE = EROWS // 16
ROWS_PER_TILE_HALF = EROWS // 32

_SC_MESH = plsc.VectorSubcoreMesh(core_axis_name="c", subcore_axis_name="s")


def _rsqrt16(d):
    i = lax.bitcast_convert_type(d, jnp.int32)
    i = jnp.int32(0x5F3759DF) - lax.shift_right_logical(i, jnp.int32(1))
    y = lax.bitcast_convert_type(i, jnp.float32)
    for _ in range(3):
        y = y * (1.5 - 0.5 * d * y * y)
    return y


@functools.partial(
    pl.kernel,
    out_type=(
        jax.ShapeDtypeStruct((NP,), jnp.float32),
        jax.ShapeDtypeStruct((NP,), jnp.float32),
    ),
    mesh=_SC_MESH,
    scratch_types=[
        pltpu.VMEM((KB, 128), jnp.int32),
        pltpu.VMEM((128,), jnp.float32),
        pltpu.VMEM((TS,), jnp.float32),
        pltpu.VMEM_SHARED((NP,), jnp.float32),
        pltpu.SemaphoreType.DMA,
    ],
)
def _deg_kernel(dst_hbm, deg_a, deg_b, dbuf, ones, slbuf, acc, sem):
    c = lax.axis_index("c")
    s = lax.axis_index("s")
    off = pl.multiple_of(s * TS, 8)

    def fill(i, _):
        ix = pl.ds(pl.multiple_of(i * 16, 16), 16)
        slbuf[ix] = jnp.zeros((16,), jnp.float32)
        return 0

    lax.fori_loop(0, TS // 16, fill, 0)
    for j in range(8):
        ones[pl.ds(j * 16, 16)] = jnp.ones((16,), jnp.float32)
    pltpu.sync_copy(slbuf, acc.at[pl.ds(off, TS)])
    plsc.subcore_barrier()

    base = c * (16 * ROWS_PER_TILE_HALF) + s * ROWS_PER_TILE_HALF

    def blk(b, _):
        row0 = base + b * KB
        pltpu.sync_copy(dst_hbm.at[pl.ds(row0, KB)], dbuf)
        descs = [
            pltpu.async_copy(ones, acc.at[dbuf.at[j]], sem, add=True)
            for j in range(KB)
        ]
        for d in descs:
            d.wait()
        return 0

    lax.fori_loop(0, ROWS_PER_TILE_HALF // KB, blk, 0)
    plsc.subcore_barrier()

    @pl.when(c == 0)
    def _():
        pltpu.sync_copy(acc.at[pl.ds(off, TS)], deg_a.at[pl.ds(off, TS)])

    @pl.when(c == 1)
    def _():
        pltpu.sync_copy(acc.at[pl.ds(off, TS)], deg_b.at[pl.ds(off, TS)])


@functools.partial(
    pl.kernel,
    out_type=(
        jax.ShapeDtypeStruct((NP,), jnp.float32),
        jax.ShapeDtypeStruct((NP,), jnp.float32),
        jax.ShapeDtypeStruct((NP,), jnp.float32),
    ),
    mesh=_SC_MESH,
    scratch_types=[
        pltpu.VMEM((KB, 128), jnp.int32),
        pltpu.VMEM((KB, 128), jnp.int32),
        pltpu.VMEM((KB, 128), jnp.float32),
        pltpu.VMEM((TS,), jnp.float32),
        pltpu.VMEM((TS,), jnp.float32),
        pltpu.VMEM((TS,), jnp.float32),
        pltpu.VMEM_SHARED((NP,), jnp.float32),
        pltpu.VMEM_SHARED((NP,), jnp.float32),
        pltpu.VMEM_SHARED((NP,), jnp.float32),
        pltpu.VMEM_SHARED((NP,), jnp.float32),
        pltpu.SemaphoreType.DMA,
    ],
)
def _chain_kernel(src_hbm, dst_hbm, h0_hbm, deg_a, deg_b, s1o, s2o, v3o,
                  srcb, dstb, vals, t_a, t_b, t_c, w_sp, acc_sp, q_sp,
                  y_sp, sem):
    c = lax.axis_index("c")
    s = lax.axis_index("s")
    off = pl.multiple_of(s * TS, 8)
    sl = pl.ds(off, TS)

    pltpu.sync_copy(deg_a.at[sl], t_a)
    pltpu.sync_copy(deg_b.at[sl], t_b)

    @pl.when(c == 1)
    def _():
        pltpu.sync_copy(h0_hbm.at[sl], t_c)

    def ew0(i, _):
        ix = pl.ds(pl.multiple_of(i * 16, 16), 16)
        d = t_a[ix] + t_b[ix] + 1.0
        y = _rsqrt16(d)
        t_a[ix] = y
        t_b[ix] = y * y
        return 0

    lax.fori_loop(0, TS // 16, ew0, 0)
    pltpu.sync_copy(t_a, y_sp.at[sl])
    pltpu.sync_copy(t_b, q_sp.at[sl])

    @pl.when(c == 0)
    def _():
        pltpu.sync_copy(t_a, w_sp.at[sl])
        pltpu.sync_copy(t_a, acc_sp.at[sl])

    @pl.when(c == 1)
    def _():
        def mul(i, _):
            ix = pl.ds(pl.multiple_of(i * 16, 16), 16)
            t_c[ix] = t_c[ix] * t_a[ix]
            return 0

        lax.fori_loop(0, TS // 16, mul, 0)
        pltpu.sync_copy(t_c, w_sp.at[sl])
        pltpu.sync_copy(t_c, acc_sp.at[sl])

    plsc.subcore_barrier()

    ebase = s * ROWS_PER_TILE

    def edge_pass():
        def blk(b, _):
            row0 = ebase + b * KB
            pltpu.sync_copy(src_hbm.at[pl.ds(row0, KB)], srcb)
            pltpu.sync_copy(dst_hbm.at[pl.ds(row0, KB)], dstb)
            descs = [
                pltpu.async_copy(w_sp.at[srcb.at[j]], vals.at[j], sem)
                for j in range(KB)
            ]
            for d in descs:
                d.wait()
            descs = [
                pltpu.async_copy(vals.at[j], acc_sp.at[dstb.at[j]], sem,
                                 add=True)
                for j in range(KB)
            ]
            for d in descs:
                d.wait()
            return 0

        lax.fori_loop(0, ROWS_PER_TILE // KB, blk, 0)

    def ew(out_ref, cont):
        pltpu.sync_copy(acc_sp.at[sl], t_a)
        pltpu.sync_copy(q_sp.at[sl], t_b)
        if out_ref is not None:
            pltpu.sync_copy(y_sp.at[sl], t_c)

        def body(i, _):
            ix = pl.ds(pl.multiple_of(i * 16, 16), 16)
            a = t_a[ix]
            if out_ref is not None:
                t_c[ix] = a * t_c[ix]
            if cont:
                t_a[ix] = a * t_b[ix]
            return 0

        lax.fori_loop(0, TS // 16, body, 0)
        if out_ref is not None:
            pltpu.sync_copy(t_c, out_ref.at[sl])
        if cont:
            pltpu.sync_copy(t_a, w_sp.at[sl])
            pltpu.sync_copy(t_a, acc_sp.at[sl])

    @pl.when(c == 0)
    def _():
        edge_pass()
        plsc.subcore_barrier()
        ew(s1o, True)
        plsc.subcore_barrier()
        edge_pass()
        plsc.subcore_barrier()
        ew(s2o, False)
        plsc.subcore_barrier()
        plsc.subcore_barrier()

    @pl.when(c == 1)
    def _():
        edge_pass()
        plsc.subcore_barrier()
        ew(None, True)
        plsc.subcore_barrier()
        edge_pass()
        plsc.subcore_barrier()
        ew(None, True)
        plsc.subcore_barrier()
        edge_pass()
        plsc.subcore_barrier()
        ew(v3o, False)


def _enc_body(x_ref, wi_ref, bi_ref, w2_ref, w3_ref, u1_ref, u2_ref,
              u3_ref, wl1_ref, blr_ref, h0_ref, p4_ref):
    f32 = jnp.float32
    h = jnp.dot(x_ref[...], wi_ref[...], preferred_element_type=f32)
    h0_ref[...] = jax.nn.leaky_relu(h + bi_ref[...])
    t = jnp.dot(u1_ref[...], w2_ref[...], preferred_element_type=f32)
    g = (jnp.dot(t, w3_ref[...], preferred_element_type=f32)
         + jnp.dot(u2_ref[...], w3_ref[...], preferred_element_type=f32)
         + u3_ref[...])
    p4_ref[...] = (jnp.dot(g, wl1_ref[...], preferred_element_type=f32)
                   + blr_ref[...])


def _enc_call(x, wi, bi, w2p, w3p, u1, u2, u3, wl1p, blr):
    return pl.pallas_call(
        _enc_body,
        out_shape=(
            jax.ShapeDtypeStruct((1000, 100), jnp.float32),
            jax.ShapeDtypeStruct((8, 512), jnp.float32),
        ),
    )(x, wi, bi, w2p, w3p, u1, u2, u3, wl1p, blr)


def _dec_body(s4_ref, p4_ref, wl2_ref, bl2_ref, wl3_ref, bl3_ref, out_ref):
    f32 = jnp.float32
    z = jnp.dot(s4_ref[...], p4_ref[...], preferred_element_type=f32)
    z = jax.nn.leaky_relu(z)
    g = jnp.dot(z, wl2_ref[...], preferred_element_type=f32) + bl2_ref[...]
    g = jax.nn.leaky_relu(g)
    o = jnp.dot(g, wl3_ref[...], preferred_element_type=f32) + bl3_ref[...]
    out_ref[...] = jax.nn.leaky_relu(o)


def _dec_call(s4, p4, wl2, bl2, wl3, bl3):
    rows = 2000
    grid = (N // rows,)
    return pl.pallas_call(
        _dec_body,
        grid=grid,
        in_specs=[
            pl.BlockSpec((rows, 8), lambda i: (i, 0)),
            pl.BlockSpec((8, 512), lambda i: (0, 0)),
            pl.BlockSpec((512, 128), lambda i: (0, 0)),
            pl.BlockSpec((1, 128), lambda i: (0, 0)),
            pl.BlockSpec((128, 512), lambda i: (0, 0)),
            pl.BlockSpec((1, 512), lambda i: (0, 0)),
        ],
        out_specs=pl.BlockSpec((rows, 512), lambda i: (i, 0)),
        out_shape=jax.ShapeDtypeStruct((N, 512), jnp.float32),
    )(s4, p4, wl2, bl2, wl3, bl3)


def kernel(x, edge_index, W_inv, b_inv, W1, b1, W2, b2, W3, b3, Wl1, bl1,
           Wl2, bl2, Wl3, bl3):
    f32 = jnp.float32

    pad_idx = N + (jnp.arange(NEP - NE, dtype=jnp.int32) % (NP - N))
    srcp = jnp.concatenate([edge_index[0], pad_idx]).reshape(EROWS, 128)
    dstp = jnp.concatenate([edge_index[1], pad_idx]).reshape(EROWS, 128)

    w2p = jnp.zeros((16, 16), f32).at[:9, :3].set(W2)
    w3p = jnp.zeros((16, 16), f32).at[:3, :3].set(W3)
    u1 = jnp.zeros((8, 16), f32).at[0, :9].set(W1[0]).at[1, :9].set(b1)
    u2 = jnp.zeros((8, 16), f32).at[2, :3].set(b2)
    u3 = jnp.zeros((8, 16), f32).at[3, :3].set(b3)
    wl1p = jnp.zeros((16, 512), f32).at[:3].set(Wl1)
    blr = jnp.zeros((8, 512), f32).at[3].set(bl1)

    h0m, p4 = _enc_call(x, W_inv, b_inv.reshape(1, 100), w2p, w3p, u1, u2,
                        u3, wl1p, blr)
    h0p = jnp.concatenate([h0m.reshape(-1), jnp.zeros((NP - N,), f32)])

    deg_a, deg_b = _deg_kernel(dstp)
    s1, s2, v3 = _chain_kernel(srcp, dstp, h0p, deg_a, deg_b)

    s4 = jnp.stack(
        [v3[:N], s2[:N], s1[:N], jnp.ones((N,), f32)], axis=1)
    s4 = jnp.concatenate([s4, jnp.zeros((N, 4), f32)], axis=1)

    out = _dec_call(s4, p4, Wl2, bl2.reshape(1, 128), Wl3,
                    bl3.reshape(1, 512))
    return out, edge_index


# speedup vs baseline: 89.4187x; 1.0752x over previous
import functools

import jax
import jax.numpy as jnp
from jax import lax
from jax.experimental import pallas as pl
from jax.experimental.pallas import tpu as pltpu
from jax.experimental.pallas import tpu_sc as plsc

N = 100000
NE = 3200000
NP = 100352
TS = NP // 16
EROWS = 25088
NEP = EROWS * 128
KB = 16
ROWS_PER_TILE = EROWS // 16
ROWS_PER_TILE_HALF = EROWS // 32

_SC_MESH = plsc.VectorSubcoreMesh(core_axis_name="c", subcore_axis_name="s")


def _rsqrt16(d):
    i = lax.bitcast_convert_type(d, jnp.int32)
    i = jnp.int32(0x5F3759DF) - lax.shift_right_logical(i, jnp.int32(1))
    y = lax.bitcast_convert_type(i, jnp.float32)
    for _ in range(3):
        y = y * (1.5 - 0.5 * d * y * y)
    return y


@functools.partial(
    pl.kernel,
    out_type=(
        jax.ShapeDtypeStruct((NP,), jnp.float32),
        jax.ShapeDtypeStruct((NP,), jnp.float32),
    ),
    mesh=_SC_MESH,
    scratch_types=[
        pltpu.VMEM((KB, 128), jnp.int32),
        pltpu.VMEM((128,), jnp.float32),
        pltpu.VMEM((TS,), jnp.float32),
        pltpu.VMEM_SHARED((NP,), jnp.float32),
        pltpu.SemaphoreType.DMA,
    ],
    compiler_params=pltpu.CompilerParams(needs_layout_passes=False),
)
def _deg_kernel(dst_hbm, deg_a, deg_b, dbuf, ones, slbuf, acc, sem):
    c = lax.axis_index("c")
    s = lax.axis_index("s")
    off = pl.multiple_of(s * TS, 8)

    def fill(i, _):
        ix = pl.ds(pl.multiple_of(i * 16, 16), 16)
        slbuf[ix] = jnp.zeros((16,), jnp.float32)
        return 0

    lax.fori_loop(0, TS // 16, fill, 0)
    for j in range(8):
        ones[pl.ds(j * 16, 16)] = jnp.ones((16,), jnp.float32)
    pltpu.sync_copy(slbuf, acc.at[pl.ds(off, TS)])
    plsc.subcore_barrier()

    base = c * (16 * ROWS_PER_TILE_HALF) + s * ROWS_PER_TILE_HALF

    def blk(b, _):
        row0 = base + b * KB
        pltpu.sync_copy(dst_hbm.at[pl.ds(row0, KB)], dbuf)
        descs = [
            pltpu.async_copy(ones, acc.at[dbuf.at[j]], sem, add=True)
            for j in range(KB)
        ]
        for d in descs:
            d.wait()
        return 0

    lax.fori_loop(0, ROWS_PER_TILE_HALF // KB, blk, 0)
    plsc.subcore_barrier()

    @pl.when(c == 0)
    def _():
        pltpu.sync_copy(acc.at[pl.ds(off, TS)], deg_a.at[pl.ds(off, TS)])

    @pl.when(c == 1)
    def _():
        pltpu.sync_copy(acc.at[pl.ds(off, TS)], deg_b.at[pl.ds(off, TS)])


@functools.partial(
    pl.kernel,
    out_type=(
        jax.ShapeDtypeStruct((NP,), jnp.float32),
        jax.ShapeDtypeStruct((NP,), jnp.float32),
        jax.ShapeDtypeStruct((NP,), jnp.float32),
        jax.ShapeDtypeStruct((NP,), jnp.float32),
        jax.ShapeDtypeStruct((NP,), jnp.float32),
        jax.ShapeDtypeStruct((NP,), jnp.float32),
        jax.ShapeDtypeStruct((NP,), jnp.float32),
    ),
    mesh=_SC_MESH,
    scratch_types=[
        pltpu.VMEM((KB, 128), jnp.int32),
        pltpu.VMEM((KB, 128), jnp.int32),
        pltpu.VMEM((KB, 128), jnp.float32),
        pltpu.VMEM((TS,), jnp.float32),
        pltpu.VMEM((TS,), jnp.float32),
        pltpu.VMEM((N,), jnp.float32),
        pltpu.VMEM_SHARED((NP,), jnp.float32),
        pltpu.SemaphoreType.DMA,
    ],
    compiler_params=pltpu.CompilerParams(needs_layout_passes=False),
)
def _chain_kernel(src_hbm, dst_hbm, h0_hbm, deg_a, deg_b, s1o, s2o, v3o,
                  ws_hbm, wv_hbm, deg0, deg1, srcb, dstb, vals, t_a, t_b,
                  w_tile, acc_sp, sem):
    c = lax.axis_index("c")
    s = lax.axis_index("s")
    off = pl.multiple_of(s * TS, 8)
    sl = pl.ds(off, TS)

    pltpu.sync_copy(deg_a.at[sl], t_a)
    pltpu.sync_copy(deg_b.at[sl], t_b)

    def ew0(i, _):
        ix = pl.ds(pl.multiple_of(i * 16, 16), 16)
        t_a[ix] = t_a[ix] + t_b[ix] + 1.0
        return 0

    lax.fori_loop(0, TS // 16, ew0, 0)

    @pl.when(c == 0)
    def _():
        pltpu.sync_copy(t_a, deg0.at[sl])

    @pl.when(c == 1)
    def _():
        pltpu.sync_copy(t_a, deg1.at[sl])

    def ew1(i, _):
        ix = pl.ds(pl.multiple_of(i * 16, 16), 16)
        t_b[ix] = _rsqrt16(t_a[ix])
        return 0

    lax.fori_loop(0, TS // 16, ew1, 0)

    @pl.when(c == 0)
    def _():
        pltpu.sync_copy(t_b, ws_hbm.at[sl])
        pltpu.sync_copy(t_b, acc_sp.at[sl])

    @pl.when(c == 1)
    def _():
        pltpu.sync_copy(h0_hbm.at[sl], t_a)

        def mul(i, _):
            ix = pl.ds(pl.multiple_of(i * 16, 16), 16)
            t_a[ix] = t_a[ix] * t_b[ix]
            return 0

        lax.fori_loop(0, TS // 16, mul, 0)
        pltpu.sync_copy(t_a, wv_hbm.at[sl])
        pltpu.sync_copy(t_a, acc_sp.at[sl])

    plsc.subcore_barrier()

    ebase = s * ROWS_PER_TILE

    def edge_pass(w_hbm):
        pltpu.sync_copy(w_hbm.at[pl.ds(0, N)], w_tile)

        def blk(b, _):
            row0 = ebase + b * KB
            pltpu.sync_copy(src_hbm.at[pl.ds(row0, KB)], srcb)
            pltpu.sync_copy(dst_hbm.at[pl.ds(row0, KB)], dstb)
            for j in range(KB):
                for g in range(8):
                    ix = pl.ds(g * 16, 16)
                    vals[j, ix] = plsc.load_gather(w_tile, [srcb[j, ix]])
            descs = [
                pltpu.async_copy(vals.at[j], acc_sp.at[dstb.at[j]], sem,
                                 add=True)
                for j in range(KB)
            ]
            for d in descs:
                d.wait()
            return 0

        lax.fori_loop(0, ROWS_PER_TILE // KB, blk, 0)

    def ew(out_ref, cont, w_hbm, deg_hbm):
        pltpu.sync_copy(acc_sp.at[sl], t_a)
        pltpu.sync_copy(deg_hbm.at[sl], t_b)

        def body(i, _):
            ix = pl.ds(pl.multiple_of(i * 16, 16), 16)
            a = t_a[ix]
            y = _rsqrt16(t_b[ix])
            if out_ref is not None:
                t_b[ix] = y * a
            if cont:
                t_a[ix] = (y * y) * a
            return 0

        lax.fori_loop(0, TS // 16, body, 0)
        if out_ref is not None:
            pltpu.sync_copy(t_b, out_ref.at[sl])
        if cont:
            pltpu.sync_copy(t_a, w_hbm.at[sl])
            pltpu.sync_copy(t_a, acc_sp.at[sl])

    @pl.when(c == 0)
    def _():
        edge_pass(ws_hbm)
        plsc.subcore_barrier()
        ew(s1o, True, ws_hbm, deg0)
        plsc.subcore_barrier()
        edge_pass(ws_hbm)
        plsc.subcore_barrier()
        ew(s2o, False, ws_hbm, deg0)
        plsc.subcore_barrier()
        plsc.subcore_barrier()

    @pl.when(c == 1)
    def _():
        edge_pass(wv_hbm)
        plsc.subcore_barrier()
        ew(None, True, wv_hbm, deg1)
        plsc.subcore_barrier()
        edge_pass(wv_hbm)
        plsc.subcore_barrier()
        ew(None, True, wv_hbm, deg1)
        plsc.subcore_barrier()
        edge_pass(wv_hbm)
        plsc.subcore_barrier()
        ew(v3o, False, wv_hbm, deg1)


def _enc_body(x_ref, wi_ref, bi_ref, w2_ref, w3_ref, u1_ref, u2_ref,
              u3_ref, wl1_ref, blr_ref, h0_ref, p4_ref):
    f32 = jnp.float32
    h = jnp.dot(x_ref[...], wi_ref[...], preferred_element_type=f32)
    h0_ref[...] = jax.nn.leaky_relu(h + bi_ref[...])
    t = jnp.dot(u1_ref[...], w2_ref[...], preferred_element_type=f32)
    g = (jnp.dot(t, w3_ref[...], preferred_element_type=f32)
         + jnp.dot(u2_ref[...], w3_ref[...], preferred_element_type=f32)
         + u3_ref[...])
    p4_ref[...] = (jnp.dot(g, wl1_ref[...], preferred_element_type=f32)
                   + blr_ref[...])


def _enc_call(x, wi, bi, w2p, w3p, u1, u2, u3, wl1p, blr):
    return pl.pallas_call(
        _enc_body,
        out_shape=(
            jax.ShapeDtypeStruct((1000, 100), jnp.float32),
            jax.ShapeDtypeStruct((8, 512), jnp.float32),
        ),
    )(x, wi, bi, w2p, w3p, u1, u2, u3, wl1p, blr)


def _dec_body(s4_ref, p4_ref, wl2_ref, bl2_ref, wl3_ref, bl3_ref, out_ref):
    f32 = jnp.float32
    z = jnp.dot(s4_ref[...], p4_ref[...], preferred_element_type=f32)
    z = jax.nn.leaky_relu(z)
    g = jnp.dot(z, wl2_ref[...], preferred_element_type=f32) + bl2_ref[...]
    g = jax.nn.leaky_relu(g)
    o = jnp.dot(g, wl3_ref[...], preferred_element_type=f32) + bl3_ref[...]
    out_ref[...] = jax.nn.leaky_relu(o)


def _dec_call(s4, p4, wl2, bl2, wl3, bl3):
    rows = 2000
    grid = (N // rows,)
    return pl.pallas_call(
        _dec_body,
        grid=grid,
        in_specs=[
            pl.BlockSpec((rows, 8), lambda i: (i, 0)),
            pl.BlockSpec((8, 512), lambda i: (0, 0)),
            pl.BlockSpec((512, 128), lambda i: (0, 0)),
            pl.BlockSpec((1, 128), lambda i: (0, 0)),
            pl.BlockSpec((128, 512), lambda i: (0, 0)),
            pl.BlockSpec((1, 512), lambda i: (0, 0)),
        ],
        out_specs=pl.BlockSpec((rows, 512), lambda i: (i, 0)),
        out_shape=jax.ShapeDtypeStruct((N, 512), jnp.float32),
    )(s4, p4, wl2, bl2, wl3, bl3)


def kernel(x, edge_index, W_inv, b_inv, W1, b1, W2, b2, W3, b3, Wl1, bl1,
           Wl2, bl2, Wl3, bl3):
    f32 = jnp.float32

    pad_dst = N + (jnp.arange(NEP - NE, dtype=jnp.int32) % (NP - N))
    pad_src = jnp.zeros((NEP - NE,), jnp.int32)
    srcp = jnp.concatenate([edge_index[0], pad_src]).reshape(EROWS, 128)
    dstp = jnp.concatenate([edge_index[1], pad_dst]).reshape(EROWS, 128)

    w2p = jnp.zeros((16, 16), f32).at[:9, :3].set(W2)
    w3p = jnp.zeros((16, 16), f32).at[:3, :3].set(W3)
    u1 = jnp.zeros((8, 16), f32).at[0, :9].set(W1[0]).at[1, :9].set(b1)
    u2 = jnp.zeros((8, 16), f32).at[2, :3].set(b2)
    u3 = jnp.zeros((8, 16), f32).at[3, :3].set(b3)
    wl1p = jnp.zeros((16, 512), f32).at[:3].set(Wl1)
    blr = jnp.zeros((8, 512), f32).at[3].set(bl1)

    h0m, p4 = _enc_call(x, W_inv, b_inv.reshape(1, 100), w2p, w3p, u1, u2,
                        u3, wl1p, blr)
    h0p = jnp.concatenate([h0m.reshape(-1), jnp.zeros((NP - N,), f32)])

    deg_a, deg_b = _deg_kernel(dstp)
    s1, s2, v3 = _chain_kernel(srcp, dstp, h0p, deg_a, deg_b)[:3]

    s4 = jnp.stack(
        [v3[:N], s2[:N], s1[:N], jnp.ones((N,), f32)], axis=1)
    s4 = jnp.concatenate([s4, jnp.zeros((N, 4), f32)], axis=1)

    out = _dec_call(s4, p4, Wl2, bl2.reshape(1, 128), Wl3,
                    bl3.reshape(1, 512))
    return out, edge_index


# speedup vs baseline: 97.0076x; 1.0849x over previous
import functools

import jax
import jax.numpy as jnp
from jax import lax
from jax.experimental import pallas as pl
from jax.experimental.pallas import tpu as pltpu
from jax.experimental.pallas import tpu_sc as plsc

N = 100000
NE = 3200000
NP = 100352
TS = NP // 16
TH = TS // 2
EROWS = 25088
NEP = EROWS * 128
KB = 8
NBLK = EROWS // KB
BLK_PER_TILE = NBLK // 16
BLK_PER_TILE_HALF = NBLK // 32

_SC_MESH = plsc.VectorSubcoreMesh(core_axis_name="c", subcore_axis_name="s")


def _rsqrt16(d):
    i = lax.bitcast_convert_type(d, jnp.int32)
    i = jnp.int32(0x5F3759DF) - lax.shift_right_logical(i, jnp.int32(1))
    y = lax.bitcast_convert_type(i, jnp.float32)
    for _ in range(3):
        y = y * (1.5 - 0.5 * d * y * y)
    return y


@functools.partial(
    pl.kernel,
    out_type=(
        jax.ShapeDtypeStruct((NP,), jnp.float32),
        jax.ShapeDtypeStruct((NP,), jnp.float32),
    ),
    mesh=_SC_MESH,
    scratch_types=[
        pltpu.VMEM((KB, 128), jnp.int32),
        pltpu.VMEM((128,), jnp.float32),
        pltpu.VMEM((TS,), jnp.float32),
        pltpu.VMEM_SHARED((NP,), jnp.float32),
        pltpu.SemaphoreType.DMA,
    ],
    compiler_params=pltpu.CompilerParams(needs_layout_passes=False),
)
def _deg_kernel(sd_hbm, deg_a, deg_b, dbuf, ones, slbuf, acc, sem):
    c = lax.axis_index("c")
    s = lax.axis_index("s")
    off = pl.multiple_of(s * TS, 8)

    def fill(i, _):
        ix = pl.ds(pl.multiple_of(i * 16, 16), 16)
        slbuf[ix] = jnp.zeros((16,), jnp.float32)
        return 0

    lax.fori_loop(0, TS // 16, fill, 0)
    for j in range(8):
        ones[pl.ds(j * 16, 16)] = jnp.ones((16,), jnp.float32)
    pltpu.sync_copy(slbuf, acc.at[pl.ds(off, TS)])
    plsc.subcore_barrier()

    base = c * (16 * BLK_PER_TILE_HALF) + s * BLK_PER_TILE_HALF

    def blk(b, _):
        bidx = base + b
        pltpu.sync_copy(sd_hbm.at[bidx, pl.ds(KB, KB)], dbuf)
        descs = [
            pltpu.async_copy(ones, acc.at[dbuf.at[j]], sem, add=True)
            for j in range(KB)
        ]
        for d in descs:
            d.wait()
        return 0

    lax.fori_loop(0, BLK_PER_TILE_HALF, blk, 0)
    plsc.subcore_barrier()

    @pl.when(c == 0)
    def _():
        pltpu.sync_copy(acc.at[pl.ds(off, TS)], deg_a.at[pl.ds(off, TS)])

    @pl.when(c == 1)
    def _():
        pltpu.sync_copy(acc.at[pl.ds(off, TS)], deg_b.at[pl.ds(off, TS)])


@functools.partial(
    pl.kernel,
    out_type=(
        jax.ShapeDtypeStruct((NP,), jnp.float32),
        jax.ShapeDtypeStruct((NP,), jnp.float32),
        jax.ShapeDtypeStruct((NP,), jnp.float32),
        jax.ShapeDtypeStruct((NP,), jnp.float32),
        jax.ShapeDtypeStruct((NP,), jnp.float32),
        jax.ShapeDtypeStruct((NP,), jnp.float32),
        jax.ShapeDtypeStruct((NP,), jnp.float32),
    ),
    mesh=_SC_MESH,
    scratch_types=[
        pltpu.VMEM((2 * KB, 128), jnp.int32),
        pltpu.VMEM((2 * KB, 128), jnp.int32),
        pltpu.VMEM((2 * KB, 128), jnp.int32),
        pltpu.VMEM((2 * KB, 128), jnp.int32),
        pltpu.VMEM((KB, 128), jnp.float32),
        pltpu.VMEM((KB, 128), jnp.float32),
        pltpu.VMEM((KB, 128), jnp.float32),
        pltpu.VMEM((KB, 128), jnp.float32),
        pltpu.VMEM((TH,), jnp.float32),
        pltpu.VMEM((TH,), jnp.float32),
        pltpu.VMEM((N,), jnp.float32),
        pltpu.VMEM_SHARED((NP,), jnp.float32),
        pltpu.SemaphoreType.DMA,
        pltpu.SemaphoreType.DMA,
        pltpu.SemaphoreType.DMA,
        pltpu.SemaphoreType.DMA,
    ],
    compiler_params=pltpu.CompilerParams(needs_layout_passes=False),
)
def _chain_kernel(sd_hbm, h0_hbm, deg_a, deg_b, s1o, s2o, v3o, ws_hbm,
                  wv_hbm, deg0, deg1, sd0, sd1, sd2, sd3, va0, va1, va2,
                  va3, t_a, t_b, w_tile, acc_sp, sm0, sm1, sm2, sm3):
    c = lax.axis_index("c")
    s = lax.axis_index("s")
    off = pl.multiple_of(s * TS, 8)
    sets = ((sd0, va0, sm0), (sd1, va1, sm1), (sd2, va2, sm2),
            (sd3, va3, sm3))

    for h in range(2):
        hsl = pl.ds(pl.multiple_of(off + h * TH, 8), TH)
        pltpu.sync_copy(deg_a.at[hsl], t_a)
        pltpu.sync_copy(deg_b.at[hsl], t_b)

        def ew0(i, _):
            ix = pl.ds(pl.multiple_of(i * 16, 16), 16)
            t_a[ix] = t_a[ix] + t_b[ix] + 1.0
            return 0

        lax.fori_loop(0, TH // 16, ew0, 0)

        @pl.when(c == 0)
        def _():
            pltpu.sync_copy(t_a, deg0.at[hsl])

        @pl.when(c == 1)
        def _():
            pltpu.sync_copy(t_a, deg1.at[hsl])

        def ew1(i, _):
            ix = pl.ds(pl.multiple_of(i * 16, 16), 16)
            t_b[ix] = _rsqrt16(t_a[ix])
            return 0

        lax.fori_loop(0, TH // 16, ew1, 0)

        @pl.when(c == 0)
        def _():
            pltpu.sync_copy(t_b, ws_hbm.at[hsl])
            pltpu.sync_copy(t_b, acc_sp.at[hsl])

        @pl.when(c == 1)
        def _():
            pltpu.sync_copy(h0_hbm.at[hsl], t_a)

            def mul(i, _):
                ix = pl.ds(pl.multiple_of(i * 16, 16), 16)
                t_a[ix] = t_a[ix] * t_b[ix]
                return 0

            lax.fori_loop(0, TH // 16, mul, 0)
            pltpu.sync_copy(t_a, wv_hbm.at[hsl])
            pltpu.sync_copy(t_a, acc_sp.at[hsl])

    plsc.subcore_barrier()

    bbase = s * BLK_PER_TILE

    def edge_pass(w_hbm):
        pltpu.sync_copy(w_hbm.at[pl.ds(0, N)], w_tile)

        def it(i, _):
            b0 = bbase + 4 * i
            descs = []
            for k, (sdb, valsb, semx) in enumerate(sets):
                pltpu.sync_copy(sd_hbm.at[b0 + k], sdb)
                for j in range(KB):
                    for g in range(8):
                        ix = pl.ds(g * 16, 16)
                        valsb[j, ix] = plsc.load_gather(
                            w_tile, [sdb[j, ix]])
                for j in range(KB):
                    descs.append(
                        pltpu.async_copy(valsb.at[j],
                                         acc_sp.at[sdb.at[KB + j]], semx,
                                         add=True))
            for d in descs:
                d.wait()
            return 0

        lax.fori_loop(0, BLK_PER_TILE // 4, it, 0)

    def ew(out_ref, cont, w_hbm, deg_hbm):
        for h in range(2):
            hsl = pl.ds(pl.multiple_of(off + h * TH, 8), TH)
            pltpu.sync_copy(acc_sp.at[hsl], t_a)
            pltpu.sync_copy(deg_hbm.at[hsl], t_b)

            def body(i, _):
                ix = pl.ds(pl.multiple_of(i * 16, 16), 16)
                a = t_a[ix]
                y = _rsqrt16(t_b[ix])
                if out_ref is not None:
                    t_b[ix] = y * a
                if cont:
                    t_a[ix] = (y * y) * a
                return 0

            lax.fori_loop(0, TH // 16, body, 0)
            if out_ref is not None:
                pltpu.sync_copy(t_b, out_ref.at[hsl])
            if cont:
                pltpu.sync_copy(t_a, w_hbm.at[hsl])
                pltpu.sync_copy(t_a, acc_sp.at[hsl])

    @pl.when(c == 0)
    def _():
        edge_pass(ws_hbm)
        plsc.subcore_barrier()
        ew(s1o, True, ws_hbm, deg0)
        plsc.subcore_barrier()
        edge_pass(ws_hbm)
        plsc.subcore_barrier()
        ew(s2o, False, ws_hbm, deg0)
        plsc.subcore_barrier()
        plsc.subcore_barrier()

    @pl.when(c == 1)
    def _():
        edge_pass(wv_hbm)
        plsc.subcore_barrier()
        ew(None, True, wv_hbm, deg1)
        plsc.subcore_barrier()
        edge_pass(wv_hbm)
        plsc.subcore_barrier()
        ew(None, True, wv_hbm, deg1)
        plsc.subcore_barrier()
        edge_pass(wv_hbm)
        plsc.subcore_barrier()
        ew(v3o, False, wv_hbm, deg1)


def _enc_body(x_ref, wi_ref, bi_ref, w2_ref, w3_ref, u1_ref, u2_ref,
              u3_ref, wl1_ref, blr_ref, h0_ref, p4_ref):
    f32 = jnp.float32
    h = jnp.dot(x_ref[...], wi_ref[...], preferred_element_type=f32)
    h0_ref[...] = jax.nn.leaky_relu(h + bi_ref[...])
    t = jnp.dot(u1_ref[...], w2_ref[...], preferred_element_type=f32)
    g = (jnp.dot(t, w3_ref[...], preferred_element_type=f32)
         + jnp.dot(u2_ref[...], w3_ref[...], preferred_element_type=f32)
         + u3_ref[...])
    p4_ref[...] = (jnp.dot(g, wl1_ref[...], preferred_element_type=f32)
                   + blr_ref[...])


def _enc_call(x, wi, bi, w2p, w3p, u1, u2, u3, wl1p, blr):
    return pl.pallas_call(
        _enc_body,
        out_shape=(
            jax.ShapeDtypeStruct((1000, 100), jnp.float32),
            jax.ShapeDtypeStruct((8, 512), jnp.float32),
        ),
    )(x, wi, bi, w2p, w3p, u1, u2, u3, wl1p, blr)


def _dec_body(s4_ref, p4_ref, wl2_ref, bl2_ref, wl3_ref, bl3_ref, out_ref):
    f32 = jnp.float32
    z = jnp.dot(s4_ref[...], p4_ref[...], preferred_element_type=f32)
    z = jax.nn.leaky_relu(z)
    g = jnp.dot(z, wl2_ref[...], preferred_element_type=f32) + bl2_ref[...]
    g = jax.nn.leaky_relu(g)
    o = jnp.dot(g, wl3_ref[...], preferred_element_type=f32) + bl3_ref[...]
    out_ref[...] = jax.nn.leaky_relu(o)


def _dec_call(s4, p4, wl2, bl2, wl3, bl3):
    rows = 2000
    grid = (N // rows,)
    return pl.pallas_call(
        _dec_body,
        grid=grid,
        in_specs=[
            pl.BlockSpec((rows, 8), lambda i: (i, 0)),
            pl.BlockSpec((8, 512), lambda i: (0, 0)),
            pl.BlockSpec((512, 128), lambda i: (0, 0)),
            pl.BlockSpec((1, 128), lambda i: (0, 0)),
            pl.BlockSpec((128, 512), lambda i: (0, 0)),
            pl.BlockSpec((1, 512), lambda i: (0, 0)),
        ],
        out_specs=pl.BlockSpec((rows, 512), lambda i: (i, 0)),
        out_shape=jax.ShapeDtypeStruct((N, 512), jnp.float32),
    )(s4, p4, wl2, bl2, wl3, bl3)


def kernel(x, edge_index, W_inv, b_inv, W1, b1, W2, b2, W3, b3, Wl1, bl1,
           Wl2, bl2, Wl3, bl3):
    f32 = jnp.float32

    pad_dst = N + (jnp.arange(NEP - NE, dtype=jnp.int32) % (NP - N))
    pad_src = jnp.zeros((NEP - NE,), jnp.int32)
    srcp = jnp.concatenate([edge_index[0], pad_src]).reshape(NBLK, KB, 128)
    dstp = jnp.concatenate([edge_index[1], pad_dst]).reshape(NBLK, KB, 128)
    sd = jnp.concatenate([srcp, dstp], axis=1)

    w2p = jnp.zeros((16, 16), f32).at[:9, :3].set(W2)
    w3p = jnp.zeros((16, 16), f32).at[:3, :3].set(W3)
    u1 = jnp.zeros((8, 16), f32).at[0, :9].set(W1[0]).at[1, :9].set(b1)
    u2 = jnp.zeros((8, 16), f32).at[2, :3].set(b2)
    u3 = jnp.zeros((8, 16), f32).at[3, :3].set(b3)
    wl1p = jnp.zeros((16, 512), f32).at[:3].set(Wl1)
    blr = jnp.zeros((8, 512), f32).at[3].set(bl1)

    h0m, p4 = _enc_call(x, W_inv, b_inv.reshape(1, 100), w2p, w3p, u1, u2,
                        u3, wl1p, blr)
    h0p = jnp.concatenate([h0m.reshape(-1), jnp.zeros((NP - N,), f32)])

    deg_a, deg_b = _deg_kernel(sd)
    s1, s2, v3 = _chain_kernel(sd, h0p, deg_a, deg_b)[:3]

    s4 = jnp.stack(
        [v3[:N], s2[:N], s1[:N], jnp.ones((N,), f32)], axis=1)
    s4 = jnp.concatenate([s4, jnp.zeros((N, 4), f32)], axis=1)

    out = _dec_call(s4, p4, Wl2, bl2.reshape(1, 128), Wl3,
                    bl3.reshape(1, 512))
    return out, edge_index


# speedup vs baseline: 135.2343x; 1.3941x over previous
import functools

import jax
import jax.numpy as jnp
from jax import lax
from jax.experimental import pallas as pl
from jax.experimental.pallas import tpu as pltpu
from jax.experimental.pallas import tpu_sc as plsc

N = 100000
NE = 3200000
NP = 100352
TS = NP // 16
TH = TS // 2
EROWS = 25088
NEP = EROWS * 128
KB = 8
NBLK = EROWS // KB
BLK_PER_TILE = NBLK // 16
BLK_PER_TILE_HALF = NBLK // 32

_SC_MESH = plsc.VectorSubcoreMesh(core_axis_name="c", subcore_axis_name="s")


def _rsqrt16(d):
    i = lax.bitcast_convert_type(d, jnp.int32)
    i = jnp.int32(0x5F3759DF) - lax.shift_right_logical(i, jnp.int32(1))
    y = lax.bitcast_convert_type(i, jnp.float32)
    for _ in range(3):
        y = y * (1.5 - 0.5 * d * y * y)
    return y


@functools.partial(
    pl.kernel,
    out_type=(
        jax.ShapeDtypeStruct((NP,), jnp.float32),
        jax.ShapeDtypeStruct((NP,), jnp.float32),
    ),
    mesh=_SC_MESH,
    scratch_types=[
        pltpu.VMEM((KB, 128), jnp.int32),
        pltpu.VMEM((KB, 128), jnp.int32),
        pltpu.VMEM((128,), jnp.float32),
        pltpu.VMEM((TS,), jnp.float32),
        pltpu.VMEM_SHARED((NP,), jnp.float32),
        pltpu.SemaphoreType.DMA,
        pltpu.SemaphoreType.DMA,
        pltpu.SemaphoreType.DMA,
        pltpu.SemaphoreType.DMA,
    ],
    compiler_params=pltpu.CompilerParams(needs_layout_passes=False),
)
def _deg_kernel(sd_hbm, deg_a, deg_b, dbuf_a, dbuf_b, ones, slbuf, acc,
                sem_a, sem_b, semi_a, semi_b):
    c = lax.axis_index("c")
    s = lax.axis_index("s")
    off = pl.multiple_of(s * TS, 8)

    def fill(i, _):
        ix = pl.ds(pl.multiple_of(i * 16, 16), 16)
        slbuf[ix] = jnp.zeros((16,), jnp.float32)
        return 0

    lax.fori_loop(0, TS // 16, fill, 0)
    for j in range(8):
        ones[pl.ds(j * 16, 16)] = jnp.ones((16,), jnp.float32)
    pltpu.sync_copy(slbuf, acc.at[pl.ds(off, TS)])
    plsc.subcore_barrier()

    base = c * (16 * BLK_PER_TILE_HALF) + s * BLK_PER_TILE_HALF
    dsets = ((dbuf_a, sem_a, semi_a), (dbuf_b, sem_b, semi_b))

    def it(i, _):
        b0 = base + 2 * i
        idescs = [
            pltpu.async_copy(sd_hbm.at[b0 + k, pl.ds(KB, KB)], dbuf, semi)
            for k, (dbuf, _, semi) in enumerate(dsets)
        ]
        descs = []
        for k, (dbuf, semx, _) in enumerate(dsets):
            idescs[k].wait()
            for j in range(KB):
                descs.append(
                    pltpu.async_copy(ones, acc.at[dbuf.at[j]], semx,
                                     add=True))
        for d in descs:
            d.wait()
        return 0

    lax.fori_loop(0, BLK_PER_TILE_HALF // 2, it, 0)
    plsc.subcore_barrier()

    @pl.when(c == 0)
    def _():
        pltpu.sync_copy(acc.at[pl.ds(off, TS)], deg_a.at[pl.ds(off, TS)])

    @pl.when(c == 1)
    def _():
        pltpu.sync_copy(acc.at[pl.ds(off, TS)], deg_b.at[pl.ds(off, TS)])


@functools.partial(
    pl.kernel,
    out_type=(
        jax.ShapeDtypeStruct((NP,), jnp.float32),
        jax.ShapeDtypeStruct((NP,), jnp.float32),
        jax.ShapeDtypeStruct((NP,), jnp.float32),
        jax.ShapeDtypeStruct((NP,), jnp.float32),
        jax.ShapeDtypeStruct((NP,), jnp.float32),
        jax.ShapeDtypeStruct((NP,), jnp.float32),
        jax.ShapeDtypeStruct((NP,), jnp.float32),
    ),
    mesh=_SC_MESH,
    scratch_types=[
        pltpu.VMEM((2 * KB, 128), jnp.int32),
        pltpu.VMEM((2 * KB, 128), jnp.int32),
        pltpu.VMEM((2 * KB, 128), jnp.int32),
        pltpu.VMEM((2 * KB, 128), jnp.int32),
        pltpu.VMEM((KB, 128), jnp.float32),
        pltpu.VMEM((KB, 128), jnp.float32),
        pltpu.VMEM((KB, 128), jnp.float32),
        pltpu.VMEM((KB, 128), jnp.float32),
        pltpu.VMEM((TH,), jnp.float32),
        pltpu.VMEM((TH,), jnp.float32),
        pltpu.VMEM((N,), jnp.float32),
        pltpu.VMEM_SHARED((NP,), jnp.float32),
        pltpu.SemaphoreType.DMA,
        pltpu.SemaphoreType.DMA,
        pltpu.SemaphoreType.DMA,
        pltpu.SemaphoreType.DMA,
        pltpu.SemaphoreType.DMA,
        pltpu.SemaphoreType.DMA,
        pltpu.SemaphoreType.DMA,
        pltpu.SemaphoreType.DMA,
    ],
    compiler_params=pltpu.CompilerParams(needs_layout_passes=False),
)
def _chain_kernel(sd_hbm, h0_hbm, deg_a, deg_b, s1o, s2o, v3o, ws_hbm,
                  wv_hbm, deg0, deg1, sd0, sd1, sd2, sd3, va0, va1, va2,
                  va3, t_a, t_b, w_tile, acc_sp, sm0, sm1, sm2, sm3, si0,
                  si1, si2, si3):
    c = lax.axis_index("c")
    s = lax.axis_index("s")
    off = pl.multiple_of(s * TS, 8)
    sets = ((sd0, va0, sm0, si0), (sd1, va1, sm1, si1),
            (sd2, va2, sm2, si2), (sd3, va3, sm3, si3))

    for h in range(2):
        hsl = pl.ds(pl.multiple_of(off + h * TH, 8), TH)
        pltpu.sync_copy(deg_a.at[hsl], t_a)
        pltpu.sync_copy(deg_b.at[hsl], t_b)

        def ew0(i, _):
            ix = pl.ds(pl.multiple_of(i * 16, 16), 16)
            t_a[ix] = t_a[ix] + t_b[ix] + 1.0
            return 0

        lax.fori_loop(0, TH // 16, ew0, 0)

        @pl.when(c == 0)
        def _():
            pltpu.sync_copy(t_a, deg0.at[hsl])

        @pl.when(c == 1)
        def _():
            pltpu.sync_copy(t_a, deg1.at[hsl])

        def ew1(i, _):
            ix = pl.ds(pl.multiple_of(i * 16, 16), 16)
            t_b[ix] = _rsqrt16(t_a[ix])
            return 0

        lax.fori_loop(0, TH // 16, ew1, 0)

        @pl.when(c == 0)
        def _():
            pltpu.sync_copy(t_b, ws_hbm.at[hsl])
            pltpu.sync_copy(t_b, acc_sp.at[hsl])

        @pl.when(c == 1)
        def _():
            pltpu.sync_copy(h0_hbm.at[hsl], t_a)

            def mul(i, _):
                ix = pl.ds(pl.multiple_of(i * 16, 16), 16)
                t_a[ix] = t_a[ix] * t_b[ix]
                return 0

            lax.fori_loop(0, TH // 16, mul, 0)
            pltpu.sync_copy(t_a, wv_hbm.at[hsl])
            pltpu.sync_copy(t_a, acc_sp.at[hsl])

    plsc.subcore_barrier()

    bbase = s * BLK_PER_TILE

    def edge_pass(w_hbm):
        pltpu.sync_copy(w_hbm.at[pl.ds(0, N)], w_tile)

        def it(i, _):
            b0 = bbase + 4 * i
            idescs = [
                pltpu.async_copy(sd_hbm.at[b0 + k], sdb, semi)
                for k, (sdb, _, _, semi) in enumerate(sets)
            ]
            descs = []
            for k, (sdb, valsb, semx, _) in enumerate(sets):
                idescs[k].wait()
                for j in range(KB):
                    for g in range(8):
                        ix = pl.ds(g * 16, 16)
                        valsb[j, ix] = plsc.load_gather(
                            w_tile, [sdb[j, ix]])
                for j in range(KB):
                    descs.append(
                        pltpu.async_copy(valsb.at[j],
                                         acc_sp.at[sdb.at[KB + j]], semx,
                                         add=True))
            for d in descs:
                d.wait()
            return 0

        lax.fori_loop(0, BLK_PER_TILE // 4, it, 0)

    def ew(out_ref, cont, w_hbm, deg_hbm):
        for h in range(2):
            hsl = pl.ds(pl.multiple_of(off + h * TH, 8), TH)
            pltpu.sync_copy(acc_sp.at[hsl], t_a)
            pltpu.sync_copy(deg_hbm.at[hsl], t_b)

            def body(i, _):
                ix = pl.ds(pl.multiple_of(i * 16, 16), 16)
                a = t_a[ix]
                y = _rsqrt16(t_b[ix])
                if out_ref is not None:
                    t_b[ix] = y * a
                if cont:
                    t_a[ix] = (y * y) * a
                return 0

            lax.fori_loop(0, TH // 16, body, 0)
            if out_ref is not None:
                pltpu.sync_copy(t_b, out_ref.at[hsl])
            if cont:
                pltpu.sync_copy(t_a, w_hbm.at[hsl])
                pltpu.sync_copy(t_a, acc_sp.at[hsl])

    @pl.when(c == 0)
    def _():
        edge_pass(ws_hbm)
        plsc.subcore_barrier()
        ew(s1o, True, ws_hbm, deg0)
        plsc.subcore_barrier()
        edge_pass(ws_hbm)
        plsc.subcore_barrier()
        ew(s2o, False, ws_hbm, deg0)
        plsc.subcore_barrier()
        plsc.subcore_barrier()

    @pl.when(c == 1)
    def _():
        edge_pass(wv_hbm)
        plsc.subcore_barrier()
        ew(None, True, wv_hbm, deg1)
        plsc.subcore_barrier()
        edge_pass(wv_hbm)
        plsc.subcore_barrier()
        ew(None, True, wv_hbm, deg1)
        plsc.subcore_barrier()
        edge_pass(wv_hbm)
        plsc.subcore_barrier()
        ew(v3o, False, wv_hbm, deg1)


def _enc_body(x_ref, wi_ref, bi_ref, w2_ref, w3_ref, u1_ref, u2_ref,
              u3_ref, wl1_ref, blr_ref, h0_ref, p4_ref):
    f32 = jnp.float32
    h = jnp.dot(x_ref[...], wi_ref[...], preferred_element_type=f32)
    h0_ref[...] = jax.nn.leaky_relu(h + bi_ref[...])
    t = jnp.dot(u1_ref[...], w2_ref[...], preferred_element_type=f32)
    g = (jnp.dot(t, w3_ref[...], preferred_element_type=f32)
         + jnp.dot(u2_ref[...], w3_ref[...], preferred_element_type=f32)
         + u3_ref[...])
    p4_ref[...] = (jnp.dot(g, wl1_ref[...], preferred_element_type=f32)
                   + blr_ref[...])


def _enc_call(x, wi, bi, w2p, w3p, u1, u2, u3, wl1p, blr):
    return pl.pallas_call(
        _enc_body,
        out_shape=(
            jax.ShapeDtypeStruct((1000, 100), jnp.float32),
            jax.ShapeDtypeStruct((8, 512), jnp.float32),
        ),
    )(x, wi, bi, w2p, w3p, u1, u2, u3, wl1p, blr)


def _dec_body(s4_ref, p4_ref, wl2_ref, bl2_ref, wl3_ref, bl3_ref, out_ref):
    f32 = jnp.float32
    z = jnp.dot(s4_ref[...], p4_ref[...], preferred_element_type=f32)
    z = jax.nn.leaky_relu(z)
    g = jnp.dot(z, wl2_ref[...], preferred_element_type=f32) + bl2_ref[...]
    g = jax.nn.leaky_relu(g)
    o = jnp.dot(g, wl3_ref[...], preferred_element_type=f32) + bl3_ref[...]
    out_ref[...] = jax.nn.leaky_relu(o)


def _dec_call(s4, p4, wl2, bl2, wl3, bl3):
    rows = 2000
    grid = (N // rows,)
    return pl.pallas_call(
        _dec_body,
        grid=grid,
        in_specs=[
            pl.BlockSpec((rows, 8), lambda i: (i, 0)),
            pl.BlockSpec((8, 512), lambda i: (0, 0)),
            pl.BlockSpec((512, 128), lambda i: (0, 0)),
            pl.BlockSpec((1, 128), lambda i: (0, 0)),
            pl.BlockSpec((128, 512), lambda i: (0, 0)),
            pl.BlockSpec((1, 512), lambda i: (0, 0)),
        ],
        out_specs=pl.BlockSpec((rows, 512), lambda i: (i, 0)),
        out_shape=jax.ShapeDtypeStruct((N, 512), jnp.float32),
    )(s4, p4, wl2, bl2, wl3, bl3)


def kernel(x, edge_index, W_inv, b_inv, W1, b1, W2, b2, W3, b3, Wl1, bl1,
           Wl2, bl2, Wl3, bl3):
    f32 = jnp.float32

    pad_dst = N + (jnp.arange(NEP - NE, dtype=jnp.int32) % (NP - N))
    pad_src = jnp.zeros((NEP - NE,), jnp.int32)
    srcp = jnp.concatenate([edge_index[0], pad_src]).reshape(NBLK, KB, 128)
    dstp = jnp.concatenate([edge_index[1], pad_dst]).reshape(NBLK, KB, 128)
    sd = jnp.concatenate([srcp, dstp], axis=1)

    w2p = jnp.zeros((16, 16), f32).at[:9, :3].set(W2)
    w3p = jnp.zeros((16, 16), f32).at[:3, :3].set(W3)
    u1 = jnp.zeros((8, 16), f32).at[0, :9].set(W1[0]).at[1, :9].set(b1)
    u2 = jnp.zeros((8, 16), f32).at[2, :3].set(b2)
    u3 = jnp.zeros((8, 16), f32).at[3, :3].set(b3)
    wl1p = jnp.zeros((16, 512), f32).at[:3].set(Wl1)
    blr = jnp.zeros((8, 512), f32).at[3].set(bl1)

    h0m, p4 = _enc_call(x, W_inv, b_inv.reshape(1, 100), w2p, w3p, u1, u2,
                        u3, wl1p, blr)
    h0p = jnp.concatenate([h0m.reshape(-1), jnp.zeros((NP - N,), f32)])

    deg_a, deg_b = _deg_kernel(sd)
    s1, s2, v3 = _chain_kernel(sd, h0p, deg_a, deg_b)[:3]

    s4 = jnp.stack(
        [v3[:N], s2[:N], s1[:N], jnp.ones((N,), f32)], axis=1)
    s4 = jnp.concatenate([s4, jnp.zeros((N, 4), f32)], axis=1)

    out = _dec_call(s4, p4, Wl2, bl2.reshape(1, 128), Wl3,
                    bl3.reshape(1, 512))
    return out, edge_index


# speedup vs baseline: 135.2548x; 1.0002x over previous
import functools

import jax
import jax.numpy as jnp
from jax import lax
from jax.experimental import pallas as pl
from jax.experimental.pallas import tpu as pltpu
from jax.experimental.pallas import tpu_sc as plsc

N = 100000
NE = 3200000
NP = 100352
TS = NP // 16
TH = TS // 2
EROWS = 25088
NEP = EROWS * 128
KB = 8
NBLK = EROWS // KB
BLK_PER_TILE = NBLK // 16
BLK_PER_TILE_HALF = NBLK // 32

_SC_MESH = plsc.VectorSubcoreMesh(core_axis_name="c", subcore_axis_name="s")


def _rsqrt16(d):
    i = lax.bitcast_convert_type(d, jnp.int32)
    i = jnp.int32(0x5F3759DF) - lax.shift_right_logical(i, jnp.int32(1))
    y = lax.bitcast_convert_type(i, jnp.float32)
    for _ in range(3):
        y = y * (1.5 - 0.5 * d * y * y)
    return y


@functools.partial(
    pl.kernel,
    out_type=(
        jax.ShapeDtypeStruct((NP,), jnp.float32),
        jax.ShapeDtypeStruct((NP,), jnp.float32),
    ),
    mesh=_SC_MESH,
    scratch_types=[
        pltpu.VMEM((KB, 128), jnp.int32),
        pltpu.VMEM((KB, 128), jnp.int32),
        pltpu.VMEM((128,), jnp.float32),
        pltpu.VMEM((TS,), jnp.float32),
        pltpu.VMEM_SHARED((NP,), jnp.float32),
        pltpu.SemaphoreType.DMA,
        pltpu.SemaphoreType.DMA,
        pltpu.SemaphoreType.DMA,
        pltpu.SemaphoreType.DMA,
    ],
    compiler_params=pltpu.CompilerParams(needs_layout_passes=False),
)
def _deg_kernel(sd_hbm, deg_a, deg_b, dbuf_a, dbuf_b, ones, slbuf, acc,
                sem_a, sem_b, semi_a, semi_b):
    c = lax.axis_index("c")
    s = lax.axis_index("s")
    off = pl.multiple_of(s * TS, 8)

    def fill(i, _):
        ix = pl.ds(pl.multiple_of(i * 16, 16), 16)
        slbuf[ix] = jnp.zeros((16,), jnp.float32)
        return 0

    lax.fori_loop(0, TS // 16, fill, 0)
    for j in range(8):
        ones[pl.ds(j * 16, 16)] = jnp.ones((16,), jnp.float32)
    pltpu.sync_copy(slbuf, acc.at[pl.ds(off, TS)])
    plsc.subcore_barrier()

    base = c * (16 * BLK_PER_TILE_HALF) + s * BLK_PER_TILE_HALF
    dsets = ((dbuf_a, sem_a, semi_a), (dbuf_b, sem_b, semi_b))

    def it(i, _):
        b0 = base + 2 * i
        idescs = [
            pltpu.async_copy(sd_hbm.at[b0 + k, pl.ds(KB, KB)], dbuf, semi)
            for k, (dbuf, _, semi) in enumerate(dsets)
        ]
        descs = []
        for k, (dbuf, semx, _) in enumerate(dsets):
            idescs[k].wait()
            for j in range(KB):
                descs.append(
                    pltpu.async_copy(ones, acc.at[dbuf.at[j]], semx,
                                     add=True))
        for d in descs:
            d.wait()
        return 0

    lax.fori_loop(0, BLK_PER_TILE_HALF // 2, it, 0)
    plsc.subcore_barrier()

    @pl.when(c == 0)
    def _():
        pltpu.sync_copy(acc.at[pl.ds(off, TS)], deg_a.at[pl.ds(off, TS)])

    @pl.when(c == 1)
    def _():
        pltpu.sync_copy(acc.at[pl.ds(off, TS)], deg_b.at[pl.ds(off, TS)])


@functools.partial(
    pl.kernel,
    out_type=(
        jax.ShapeDtypeStruct((NP,), jnp.float32),
        jax.ShapeDtypeStruct((NP,), jnp.float32),
        jax.ShapeDtypeStruct((NP,), jnp.float32),
        jax.ShapeDtypeStruct((NP,), jnp.float32),
        jax.ShapeDtypeStruct((NP,), jnp.float32),
        jax.ShapeDtypeStruct((NP,), jnp.float32),
        jax.ShapeDtypeStruct((NP,), jnp.float32),
    ),
    mesh=_SC_MESH,
    scratch_types=[
        pltpu.VMEM((2 * KB, 128), jnp.int32),
        pltpu.VMEM((2 * KB, 128), jnp.int32),
        pltpu.VMEM((2 * KB, 128), jnp.int32),
        pltpu.VMEM((2 * KB, 128), jnp.int32),
        pltpu.VMEM((KB, 128), jnp.float32),
        pltpu.VMEM((KB, 128), jnp.float32),
        pltpu.VMEM((KB, 128), jnp.float32),
        pltpu.VMEM((KB, 128), jnp.float32),
        pltpu.VMEM((TH,), jnp.float32),
        pltpu.VMEM((TH,), jnp.float32),
        pltpu.VMEM((N,), jnp.float32),
        pltpu.VMEM_SHARED((NP,), jnp.float32),
        pltpu.SemaphoreType.DMA,
        pltpu.SemaphoreType.DMA,
        pltpu.SemaphoreType.DMA,
        pltpu.SemaphoreType.DMA,
        pltpu.SemaphoreType.DMA,
        pltpu.SemaphoreType.DMA,
        pltpu.SemaphoreType.DMA,
        pltpu.SemaphoreType.DMA,
    ],
    compiler_params=pltpu.CompilerParams(needs_layout_passes=False),
)
def _chain_kernel(sd_hbm, h0_hbm, deg_a, deg_b, s1o, s2o, v3o, ws_hbm,
                  wv_hbm, deg0, deg1, sd0, sd1, sd2, sd3, va0, va1, va2,
                  va3, t_a, t_b, w_tile, acc_sp, sm0, sm1, sm2, sm3, si0,
                  si1, si2, si3):
    c = lax.axis_index("c")
    s = lax.axis_index("s")
    off = pl.multiple_of(s * TS, 8)
    sets = ((sd0, va0, sm0, si0), (sd1, va1, sm1, si1),
            (sd2, va2, sm2, si2), (sd3, va3, sm3, si3))

    for h in range(2):
        hsl = pl.ds(pl.multiple_of(off + h * TH, 8), TH)
        pltpu.sync_copy(deg_a.at[hsl], t_a)
        pltpu.sync_copy(deg_b.at[hsl], t_b)

        def ew0(i, _):
            ix = pl.ds(pl.multiple_of(i * 16, 16), 16)
            t_a[ix] = t_a[ix] + t_b[ix] + 1.0
            return 0

        lax.fori_loop(0, TH // 16, ew0, 0)

        @pl.when(c == 0)
        def _():
            pltpu.sync_copy(t_a, deg0.at[hsl])

        @pl.when(c == 1)
        def _():
            pltpu.sync_copy(t_a, deg1.at[hsl])

        def ew1(i, _):
            ix = pl.ds(pl.multiple_of(i * 16, 16), 16)
            t_b[ix] = _rsqrt16(t_a[ix])
            return 0

        lax.fori_loop(0, TH // 16, ew1, 0)

        @pl.when(c == 0)
        def _():
            pltpu.sync_copy(t_b, ws_hbm.at[hsl])
            pltpu.sync_copy(t_b, acc_sp.at[hsl])

        @pl.when(c == 1)
        def _():
            pltpu.sync_copy(h0_hbm.at[hsl], t_a)

            def mul(i, _):
                ix = pl.ds(pl.multiple_of(i * 16, 16), 16)
                t_a[ix] = t_a[ix] * t_b[ix]
                return 0

            lax.fori_loop(0, TH // 16, mul, 0)
            pltpu.sync_copy(t_a, wv_hbm.at[hsl])
            pltpu.sync_copy(t_a, acc_sp.at[hsl])

    plsc.subcore_barrier()

    bbase = s * BLK_PER_TILE

    def edge_pass(w_hbm):
        pltpu.sync_copy(w_hbm.at[pl.ds(0, N)], w_tile)

        def it(i, _):
            b0 = bbase + 4 * i
            idescs = [
                pltpu.async_copy(sd_hbm.at[b0 + k], sdb, semi)
                for k, (sdb, _, _, semi) in enumerate(sets)
            ]
            descs = []
            for k, (sdb, valsb, semx, _) in enumerate(sets):
                idescs[k].wait()
                for j in range(KB):
                    for g in range(8):
                        ix = pl.ds(g * 16, 16)
                        valsb[j, ix] = plsc.load_gather(
                            w_tile, [sdb[j, ix]])
                for j in range(KB):
                    descs.append(
                        pltpu.async_copy(valsb.at[j],
                                         acc_sp.at[sdb.at[KB + j]], semx,
                                         add=True))
            for d in descs:
                d.wait()
            return 0

        lax.fori_loop(0, BLK_PER_TILE // 4, it, 0)

    def ew(out_ref, cont, w_hbm, deg_hbm):
        for h in range(2):
            hsl = pl.ds(pl.multiple_of(off + h * TH, 8), TH)
            pltpu.sync_copy(acc_sp.at[hsl], t_a)
            pltpu.sync_copy(deg_hbm.at[hsl], t_b)

            def body(i, _):
                ix = pl.ds(pl.multiple_of(i * 16, 16), 16)
                a = t_a[ix]
                y = _rsqrt16(t_b[ix])
                if out_ref is not None:
                    t_b[ix] = y * a
                if cont:
                    t_a[ix] = (y * y) * a
                return 0

            lax.fori_loop(0, TH // 16, body, 0)
            if out_ref is not None:
                pltpu.sync_copy(t_b, out_ref.at[hsl])
            if cont:
                pltpu.sync_copy(t_a, w_hbm.at[hsl])
                pltpu.sync_copy(t_a, acc_sp.at[hsl])

    @pl.when(c == 0)
    def _():
        edge_pass(ws_hbm)
        plsc.subcore_barrier()
        ew(s1o, True, ws_hbm, deg0)
        plsc.subcore_barrier()
        edge_pass(ws_hbm)
        plsc.subcore_barrier()
        ew(s2o, False, ws_hbm, deg0)
        plsc.subcore_barrier()
        plsc.subcore_barrier()

    @pl.when(c == 1)
    def _():
        edge_pass(wv_hbm)
        plsc.subcore_barrier()
        ew(None, True, wv_hbm, deg1)
        plsc.subcore_barrier()
        edge_pass(wv_hbm)
        plsc.subcore_barrier()
        ew(None, True, wv_hbm, deg1)
        plsc.subcore_barrier()
        edge_pass(wv_hbm)
        plsc.subcore_barrier()
        ew(v3o, False, wv_hbm, deg1)


def _enc_body(x_ref, wi_ref, bi_ref, w2_ref, w3_ref, u1_ref, u2_ref,
              u3_ref, wl1_ref, blr_ref, h0_ref, p4_ref):
    f32 = jnp.float32
    h = jnp.dot(x_ref[...], wi_ref[...], preferred_element_type=f32)
    h0_ref[...] = jax.nn.leaky_relu(h + bi_ref[...])
    t = jnp.dot(u1_ref[...], w2_ref[...], preferred_element_type=f32)
    g = (jnp.dot(t, w3_ref[...], preferred_element_type=f32)
         + jnp.dot(u2_ref[...], w3_ref[...], preferred_element_type=f32)
         + u3_ref[...])
    p4_ref[...] = (jnp.dot(g, wl1_ref[...], preferred_element_type=f32)
                   + blr_ref[...])


def _enc_call(x, wi, bi, w2p, w3p, u1, u2, u3, wl1p, blr):
    return pl.pallas_call(
        _enc_body,
        out_shape=(
            jax.ShapeDtypeStruct((1000, 100), jnp.float32),
            jax.ShapeDtypeStruct((8, 512), jnp.float32),
        ),
    )(x, wi, bi, w2p, w3p, u1, u2, u3, wl1p, blr)


def _dec_body(s4_ref, p4_ref, wl2_ref, bl2_ref, wl3_ref, bl3_ref, out_ref):
    f32 = jnp.float32
    bf16 = jnp.bfloat16
    z = jnp.dot(s4_ref[...], p4_ref[...], preferred_element_type=f32)
    z = jax.nn.leaky_relu(z)
    g = jnp.dot(z.astype(bf16), wl2_ref[...].astype(bf16),
                preferred_element_type=f32) + bl2_ref[...]
    g = jax.nn.leaky_relu(g)
    o = jnp.dot(g.astype(bf16), wl3_ref[...].astype(bf16),
                preferred_element_type=f32) + bl3_ref[...]
    out_ref[...] = jax.nn.leaky_relu(o)


def _dec_call(s4, p4, wl2, bl2, wl3, bl3):
    rows = 2000
    grid = (N // rows,)
    return pl.pallas_call(
        _dec_body,
        grid=grid,
        in_specs=[
            pl.BlockSpec((rows, 8), lambda i: (i, 0)),
            pl.BlockSpec((8, 512), lambda i: (0, 0)),
            pl.BlockSpec((512, 128), lambda i: (0, 0)),
            pl.BlockSpec((1, 128), lambda i: (0, 0)),
            pl.BlockSpec((128, 512), lambda i: (0, 0)),
            pl.BlockSpec((1, 512), lambda i: (0, 0)),
        ],
        out_specs=pl.BlockSpec((rows, 512), lambda i: (i, 0)),
        out_shape=jax.ShapeDtypeStruct((N, 512), jnp.float32),
    )(s4, p4, wl2, bl2, wl3, bl3)


def kernel(x, edge_index, W_inv, b_inv, W1, b1, W2, b2, W3, b3, Wl1, bl1,
           Wl2, bl2, Wl3, bl3):
    f32 = jnp.float32

    pad_dst = N + (jnp.arange(NEP - NE, dtype=jnp.int32) % (NP - N))
    pad_src = jnp.zeros((NEP - NE,), jnp.int32)
    srcp = jnp.concatenate([edge_index[0], pad_src]).reshape(NBLK, KB, 128)
    dstp = jnp.concatenate([edge_index[1], pad_dst]).reshape(NBLK, KB, 128)
    sd = jnp.concatenate([srcp, dstp], axis=1)

    w2p = jnp.zeros((16, 16), f32).at[:9, :3].set(W2)
    w3p = jnp.zeros((16, 16), f32).at[:3, :3].set(W3)
    u1 = jnp.zeros((8, 16), f32).at[0, :9].set(W1[0]).at[1, :9].set(b1)
    u2 = jnp.zeros((8, 16), f32).at[2, :3].set(b2)
    u3 = jnp.zeros((8, 16), f32).at[3, :3].set(b3)
    wl1p = jnp.zeros((16, 512), f32).at[:3].set(Wl1)
    blr = jnp.zeros((8, 512), f32).at[3].set(bl1)

    h0m, p4 = _enc_call(x, W_inv, b_inv.reshape(1, 100), w2p, w3p, u1, u2,
                        u3, wl1p, blr)
    h0p = jnp.concatenate([h0m.reshape(-1), jnp.zeros((NP - N,), f32)])

    deg_a, deg_b = _deg_kernel(sd)
    s1, s2, v3 = _chain_kernel(sd, h0p, deg_a, deg_b)[:3]

    s4 = jnp.stack(
        [v3[:N], s2[:N], s1[:N], jnp.ones((N,), f32)], axis=1)
    s4 = jnp.concatenate([s4, jnp.zeros((N, 4), f32)], axis=1)

    out = _dec_call(s4, p4, Wl2, bl2.reshape(1, 128), Wl3,
                    bl3.reshape(1, 512))
    return out, edge_index


# speedup vs baseline: 136.2418x; 1.0073x over previous
import functools

import jax
import jax.numpy as jnp
from jax import lax
from jax.experimental import pallas as pl
from jax.experimental.pallas import tpu as pltpu
from jax.experimental.pallas import tpu_sc as plsc

N = 100000
NE = 3200000
NP = 100352
TS = NP // 16
TH = TS // 2
EROWS = 25088
NEP = EROWS * 128
KB = 8
NBLK = EROWS // KB
BLK_PER_TILE = NBLK // 16
BLK_PER_TILE_HALF = NBLK // 32

_SC_MESH = plsc.VectorSubcoreMesh(core_axis_name="c", subcore_axis_name="s")


def _rsqrt16(d):
    i = lax.bitcast_convert_type(d, jnp.int32)
    i = jnp.int32(0x5F3759DF) - lax.shift_right_logical(i, jnp.int32(1))
    y = lax.bitcast_convert_type(i, jnp.float32)
    for _ in range(3):
        y = y * (1.5 - 0.5 * d * y * y)
    return y


@functools.partial(
    pl.kernel,
    out_type=(
        jax.ShapeDtypeStruct((NP,), jnp.float32),
        jax.ShapeDtypeStruct((NP,), jnp.float32),
    ),
    mesh=_SC_MESH,
    scratch_types=[
        pltpu.VMEM((KB, 128), jnp.int32),
        pltpu.VMEM((KB, 128), jnp.int32),
        pltpu.VMEM((128,), jnp.float32),
        pltpu.VMEM((TS,), jnp.float32),
        pltpu.VMEM_SHARED((NP,), jnp.float32),
        pltpu.SemaphoreType.DMA,
        pltpu.SemaphoreType.DMA,
        pltpu.SemaphoreType.DMA,
        pltpu.SemaphoreType.DMA,
    ],
    compiler_params=pltpu.CompilerParams(needs_layout_passes=False),
)
def _deg_kernel(sd_hbm, deg_a, deg_b, dbuf_a, dbuf_b, ones, slbuf, acc,
                sem_a, sem_b, semi_a, semi_b):
    c = lax.axis_index("c")
    s = lax.axis_index("s")
    off = pl.multiple_of(s * TS, 8)

    def fill(i, _):
        ix = pl.ds(pl.multiple_of(i * 16, 16), 16)
        slbuf[ix] = jnp.zeros((16,), jnp.float32)
        return 0

    lax.fori_loop(0, TS // 16, fill, 0)
    for j in range(8):
        ones[pl.ds(j * 16, 16)] = jnp.ones((16,), jnp.float32)
    pltpu.sync_copy(slbuf, acc.at[pl.ds(off, TS)])
    plsc.subcore_barrier()

    base = c * (16 * BLK_PER_TILE_HALF) + s * BLK_PER_TILE_HALF
    dsets = ((dbuf_a, sem_a, semi_a), (dbuf_b, sem_b, semi_b))

    def it(i, _):
        b0 = base + 2 * i
        idescs = [
            pltpu.async_copy(sd_hbm.at[b0 + k, pl.ds(KB, KB)], dbuf, semi)
            for k, (dbuf, _, semi) in enumerate(dsets)
        ]
        descs = []
        for k, (dbuf, semx, _) in enumerate(dsets):
            idescs[k].wait()
            for j in range(KB):
                descs.append(
                    pltpu.async_copy(ones, acc.at[dbuf.at[j]], semx,
                                     add=True))
        for d in descs:
            d.wait()
        return 0

    lax.fori_loop(0, BLK_PER_TILE_HALF // 2, it, 0)
    plsc.subcore_barrier()

    @pl.when(c == 0)
    def _():
        pltpu.sync_copy(acc.at[pl.ds(off, TS)], deg_a.at[pl.ds(off, TS)])

    @pl.when(c == 1)
    def _():
        pltpu.sync_copy(acc.at[pl.ds(off, TS)], deg_b.at[pl.ds(off, TS)])


@functools.partial(
    pl.kernel,
    out_type=(
        jax.ShapeDtypeStruct((NP,), jnp.float32),
        jax.ShapeDtypeStruct((NP,), jnp.float32),
        jax.ShapeDtypeStruct((NP,), jnp.float32),
        jax.ShapeDtypeStruct((NP,), jnp.float32),
        jax.ShapeDtypeStruct((NP,), jnp.float32),
        jax.ShapeDtypeStruct((NP,), jnp.float32),
        jax.ShapeDtypeStruct((NP,), jnp.float32),
    ),
    mesh=_SC_MESH,
    scratch_types=[
        pltpu.VMEM((2 * KB, 128), jnp.int32),
        pltpu.VMEM((2 * KB, 128), jnp.int32),
        pltpu.VMEM((2 * KB, 128), jnp.int32),
        pltpu.VMEM((2 * KB, 128), jnp.int32),
        pltpu.VMEM((KB, 128), jnp.float32),
        pltpu.VMEM((KB, 128), jnp.float32),
        pltpu.VMEM((KB, 128), jnp.float32),
        pltpu.VMEM((KB, 128), jnp.float32),
        pltpu.VMEM((TH,), jnp.float32),
        pltpu.VMEM((TH,), jnp.float32),
        pltpu.VMEM((N,), jnp.float32),
        pltpu.VMEM_SHARED((NP,), jnp.float32),
        pltpu.SemaphoreType.DMA,
        pltpu.SemaphoreType.DMA,
        pltpu.SemaphoreType.DMA,
        pltpu.SemaphoreType.DMA,
        pltpu.SemaphoreType.DMA,
        pltpu.SemaphoreType.DMA,
        pltpu.SemaphoreType.DMA,
        pltpu.SemaphoreType.DMA,
    ],
    compiler_params=pltpu.CompilerParams(needs_layout_passes=False),
)
def _chain_kernel(sd_hbm, h0_hbm, deg_a, deg_b, s1o, s2o, v3o, ws_hbm,
                  wv_hbm, deg0, deg1, sd0, sd1, sd2, sd3, va0, va1, va2,
                  va3, t_a, t_b, w_tile, acc_sp, sm0, sm1, sm2, sm3, si0,
                  si1, si2, si3):
    c = lax.axis_index("c")
    s = lax.axis_index("s")
    off = pl.multiple_of(s * TS, 8)
    sets = ((sd0, va0, sm0, si0), (sd1, va1, sm1, si1),
            (sd2, va2, sm2, si2), (sd3, va3, sm3, si3))

    for h in range(2):
        hsl = pl.ds(pl.multiple_of(off + h * TH, 8), TH)
        pltpu.sync_copy(deg_a.at[hsl], t_a)
        pltpu.sync_copy(deg_b.at[hsl], t_b)

        def ew0(i, _):
            ix = pl.ds(pl.multiple_of(i * 16, 16), 16)
            t_a[ix] = t_a[ix] + t_b[ix] + 1.0
            return 0

        lax.fori_loop(0, TH // 16, ew0, 0)

        @pl.when(c == 0)
        def _():
            pltpu.sync_copy(t_a, deg0.at[hsl])

        @pl.when(c == 1)
        def _():
            pltpu.sync_copy(t_a, deg1.at[hsl])

        def ew1(i, _):
            ix = pl.ds(pl.multiple_of(i * 16, 16), 16)
            t_b[ix] = _rsqrt16(t_a[ix])
            return 0

        lax.fori_loop(0, TH // 16, ew1, 0)

        @pl.when(c == 0)
        def _():
            pltpu.sync_copy(t_b, ws_hbm.at[hsl])
            pltpu.sync_copy(t_b, acc_sp.at[hsl])

        @pl.when(c == 1)
        def _():
            pltpu.sync_copy(h0_hbm.at[hsl], t_a)

            def mul(i, _):
                ix = pl.ds(pl.multiple_of(i * 16, 16), 16)
                t_a[ix] = t_a[ix] * t_b[ix]
                return 0

            lax.fori_loop(0, TH // 16, mul, 0)
            pltpu.sync_copy(t_a, wv_hbm.at[hsl])
            pltpu.sync_copy(t_a, acc_sp.at[hsl])

    plsc.subcore_barrier()

    bbase = s * BLK_PER_TILE

    def edge_pass(w_hbm):
        pltpu.sync_copy(w_hbm.at[pl.ds(0, N)], w_tile)

        def it(i, _):
            b0 = bbase + 4 * i
            idescs = [
                pltpu.async_copy(sd_hbm.at[b0 + k], sdb, semi)
                for k, (sdb, _, _, semi) in enumerate(sets)
            ]
            descs = []
            for k, (sdb, valsb, semx, _) in enumerate(sets):
                idescs[k].wait()
                for j in range(KB):
                    for g in range(8):
                        ix = pl.ds(g * 16, 16)
                        valsb[j, ix] = plsc.load_gather(
                            w_tile, [sdb[j, ix]])
                for j in range(KB):
                    descs.append(
                        pltpu.async_copy(valsb.at[j],
                                         acc_sp.at[sdb.at[KB + j]], semx,
                                         add=True))
            for d in descs:
                d.wait()
            return 0

        lax.fori_loop(0, BLK_PER_TILE // 4, it, 0)

    def ew(out_ref, cont, w_hbm, deg_hbm):
        for h in range(2):
            hsl = pl.ds(pl.multiple_of(off + h * TH, 8), TH)
            pltpu.sync_copy(acc_sp.at[hsl], t_a)
            pltpu.sync_copy(deg_hbm.at[hsl], t_b)

            def body(i, _):
                ix = pl.ds(pl.multiple_of(i * 16, 16), 16)
                a = t_a[ix]
                y = _rsqrt16(t_b[ix])
                if out_ref is not None:
                    t_b[ix] = y * a
                if cont:
                    t_a[ix] = (y * y) * a
                return 0

            lax.fori_loop(0, TH // 16, body, 0)
            if out_ref is not None:
                pltpu.sync_copy(t_b, out_ref.at[hsl])
            if cont:
                pltpu.sync_copy(t_a, w_hbm.at[hsl])
                pltpu.sync_copy(t_a, acc_sp.at[hsl])

    @pl.when(c == 0)
    def _():
        edge_pass(ws_hbm)
        plsc.subcore_barrier()
        ew(s1o, True, ws_hbm, deg0)
        plsc.subcore_barrier()
        edge_pass(ws_hbm)
        plsc.subcore_barrier()
        ew(s2o, False, ws_hbm, deg0)
        plsc.subcore_barrier()
        plsc.subcore_barrier()

    @pl.when(c == 1)
    def _():
        edge_pass(wv_hbm)
        plsc.subcore_barrier()
        ew(None, True, wv_hbm, deg1)
        plsc.subcore_barrier()
        edge_pass(wv_hbm)
        plsc.subcore_barrier()
        ew(None, True, wv_hbm, deg1)
        plsc.subcore_barrier()
        edge_pass(wv_hbm)
        plsc.subcore_barrier()
        ew(v3o, False, wv_hbm, deg1)


def _enc_body(x_ref, wi_ref, bi_ref, w2_ref, w3_ref, u1_ref, u2_ref,
              u3_ref, wl1_ref, blr_ref, h0_ref, p4_ref):
    f32 = jnp.float32
    h = jnp.dot(x_ref[...], wi_ref[...], preferred_element_type=f32)
    h0_ref[...] = jax.nn.leaky_relu(h + bi_ref[...])
    t = jnp.dot(u1_ref[...], w2_ref[...], preferred_element_type=f32)
    g = (jnp.dot(t, w3_ref[...], preferred_element_type=f32)
         + jnp.dot(u2_ref[...], w3_ref[...], preferred_element_type=f32)
         + u3_ref[...])
    p4_ref[...] = (jnp.dot(g, wl1_ref[...], preferred_element_type=f32)
                   + blr_ref[...])


def _enc_call(x, wi, bi, w2p, w3p, u1, u2, u3, wl1p, blr):
    return pl.pallas_call(
        _enc_body,
        out_shape=(
            jax.ShapeDtypeStruct((1000, 100), jnp.float32),
            jax.ShapeDtypeStruct((8, 512), jnp.float32),
        ),
    )(x, wi, bi, w2p, w3p, u1, u2, u3, wl1p, blr)


def _dec_body(s4_ref, p4_ref, wl2_ref, bl2_ref, wl3_ref, bl3_ref, out_ref):
    f32 = jnp.float32
    z = jnp.dot(s4_ref[...], p4_ref[...], preferred_element_type=f32)
    z = jax.nn.leaky_relu(z)
    g = jnp.dot(z, wl2_ref[...], preferred_element_type=f32) + bl2_ref[...]
    g = jax.nn.leaky_relu(g)
    o = jnp.dot(g, wl3_ref[...], preferred_element_type=f32) + bl3_ref[...]
    out_ref[...] = jax.nn.leaky_relu(o)


def _dec_call(s4, p4, wl2, bl2, wl3, bl3):
    rows = 4000
    grid = (N // rows,)
    return pl.pallas_call(
        _dec_body,
        grid=grid,
        in_specs=[
            pl.BlockSpec((rows, 8), lambda i: (i, 0)),
            pl.BlockSpec((8, 512), lambda i: (0, 0)),
            pl.BlockSpec((512, 128), lambda i: (0, 0)),
            pl.BlockSpec((1, 128), lambda i: (0, 0)),
            pl.BlockSpec((128, 512), lambda i: (0, 0)),
            pl.BlockSpec((1, 512), lambda i: (0, 0)),
        ],
        out_specs=pl.BlockSpec((rows, 512), lambda i: (i, 0)),
        out_shape=jax.ShapeDtypeStruct((N, 512), jnp.float32),
    )(s4, p4, wl2, bl2, wl3, bl3)


def kernel(x, edge_index, W_inv, b_inv, W1, b1, W2, b2, W3, b3, Wl1, bl1,
           Wl2, bl2, Wl3, bl3):
    f32 = jnp.float32

    pad_dst = N + (jnp.arange(NEP - NE, dtype=jnp.int32) % (NP - N))
    pad_src = jnp.zeros((NEP - NE,), jnp.int32)
    srcp = jnp.concatenate([edge_index[0], pad_src]).reshape(NBLK, KB, 128)
    dstp = jnp.concatenate([edge_index[1], pad_dst]).reshape(NBLK, KB, 128)
    sd = jnp.concatenate([srcp, dstp], axis=1)

    w2p = jnp.zeros((16, 16), f32).at[:9, :3].set(W2)
    w3p = jnp.zeros((16, 16), f32).at[:3, :3].set(W3)
    u1 = jnp.zeros((8, 16), f32).at[0, :9].set(W1[0]).at[1, :9].set(b1)
    u2 = jnp.zeros((8, 16), f32).at[2, :3].set(b2)
    u3 = jnp.zeros((8, 16), f32).at[3, :3].set(b3)
    wl1p = jnp.zeros((16, 512), f32).at[:3].set(Wl1)
    blr = jnp.zeros((8, 512), f32).at[3].set(bl1)

    h0m, p4 = _enc_call(x, W_inv, b_inv.reshape(1, 100), w2p, w3p, u1, u2,
                        u3, wl1p, blr)
    h0p = jnp.concatenate([h0m.reshape(-1), jnp.zeros((NP - N,), f32)])

    deg_a, deg_b = _deg_kernel(sd)
    s1, s2, v3 = _chain_kernel(sd, h0p, deg_a, deg_b)[:3]

    s4 = jnp.stack(
        [v3[:N], s2[:N], s1[:N], jnp.ones((N,), f32)], axis=1)
    s4 = jnp.concatenate([s4, jnp.zeros((N, 4), f32)], axis=1)

    out = _dec_call(s4, p4, Wl2, bl2.reshape(1, 128), Wl3,
                    bl3.reshape(1, 512))
    return out, edge_index


# speedup vs baseline: 139.6154x; 1.0248x over previous
import functools

import jax
import jax.numpy as jnp
from jax import lax
from jax.experimental import pallas as pl
from jax.experimental.pallas import tpu as pltpu
from jax.experimental.pallas import tpu_sc as plsc

N = 100000
NE = 3200000
NP = 100352
TS = NP // 16
TH = TS // 2
EROWS = 25088
NEP = EROWS * 128
KB = 8
NBLK = EROWS // KB
BLK_PER_TILE = NBLK // 16
BLK_PER_TILE_HALF = NBLK // 32

_SC_MESH = plsc.VectorSubcoreMesh(core_axis_name="c", subcore_axis_name="s")


def _rsqrt16(d):
    i = lax.bitcast_convert_type(d, jnp.int32)
    i = jnp.int32(0x5F3759DF) - lax.shift_right_logical(i, jnp.int32(1))
    y = lax.bitcast_convert_type(i, jnp.float32)
    for _ in range(3):
        y = y * (1.5 - 0.5 * d * y * y)
    return y


@functools.partial(
    pl.kernel,
    out_type=(
        jax.ShapeDtypeStruct((NP,), jnp.float32),
        jax.ShapeDtypeStruct((NP,), jnp.float32),
    ),
    mesh=_SC_MESH,
    scratch_types=[
        pltpu.VMEM((KB, 128), jnp.int32),
        pltpu.VMEM((KB, 128), jnp.int32),
        pltpu.VMEM((128,), jnp.float32),
        pltpu.VMEM((TS,), jnp.float32),
        pltpu.VMEM_SHARED((NP,), jnp.float32),
        pltpu.SemaphoreType.DMA,
        pltpu.SemaphoreType.DMA,
        pltpu.SemaphoreType.DMA,
        pltpu.SemaphoreType.DMA,
    ],
    compiler_params=pltpu.CompilerParams(needs_layout_passes=False),
)
def _deg_kernel(sd_hbm, deg_a, deg_b, dbuf_a, dbuf_b, ones, slbuf, acc,
                sem_a, sem_b, semi_a, semi_b):
    c = lax.axis_index("c")
    s = lax.axis_index("s")
    off = pl.multiple_of(s * TS, 8)

    def fill(i, _):
        ix = pl.ds(pl.multiple_of(i * 16, 16), 16)
        slbuf[ix] = jnp.zeros((16,), jnp.float32)
        return 0

    lax.fori_loop(0, TS // 16, fill, 0)
    for j in range(8):
        ones[pl.ds(j * 16, 16)] = jnp.ones((16,), jnp.float32)
    pltpu.sync_copy(slbuf, acc.at[pl.ds(off, TS)])
    plsc.subcore_barrier()

    base = c * (16 * BLK_PER_TILE_HALF) + s * BLK_PER_TILE_HALF
    dsets = ((dbuf_a, sem_a, semi_a), (dbuf_b, sem_b, semi_b))

    def it(i, _):
        b0 = base + 2 * i
        idescs = [
            pltpu.async_copy(sd_hbm.at[b0 + k, pl.ds(KB, KB)], dbuf, semi)
            for k, (dbuf, _, semi) in enumerate(dsets)
        ]
        descs = []
        for k, (dbuf, semx, _) in enumerate(dsets):
            idescs[k].wait()
            for j in range(KB):
                descs.append(
                    pltpu.async_copy(ones, acc.at[dbuf.at[j]], semx,
                                     add=True))
        for d in descs:
            d.wait()
        return 0

    lax.fori_loop(0, BLK_PER_TILE_HALF // 2, it, 0)
    plsc.subcore_barrier()

    @pl.when(c == 0)
    def _():
        pltpu.sync_copy(acc.at[pl.ds(off, TS)], deg_a.at[pl.ds(off, TS)])

    @pl.when(c == 1)
    def _():
        pltpu.sync_copy(acc.at[pl.ds(off, TS)], deg_b.at[pl.ds(off, TS)])


@functools.partial(
    pl.kernel,
    out_type=(
        jax.ShapeDtypeStruct((NP,), jnp.float32),
        jax.ShapeDtypeStruct((NP,), jnp.float32),
        jax.ShapeDtypeStruct((NP,), jnp.float32),
        jax.ShapeDtypeStruct((NP,), jnp.float32),
        jax.ShapeDtypeStruct((NP,), jnp.float32),
        jax.ShapeDtypeStruct((NP,), jnp.float32),
    ),
    mesh=_SC_MESH,
    scratch_types=[
        pltpu.VMEM((2 * KB, 128), jnp.int32),
        pltpu.VMEM((2 * KB, 128), jnp.int32),
        pltpu.VMEM((2 * KB, 128), jnp.int32),
        pltpu.VMEM((2 * KB, 128), jnp.int32),
        pltpu.VMEM((KB, 128), jnp.float32),
        pltpu.VMEM((KB, 128), jnp.float32),
        pltpu.VMEM((KB, 128), jnp.float32),
        pltpu.VMEM((KB, 128), jnp.float32),
        pltpu.VMEM((TH,), jnp.float32),
        pltpu.VMEM((TH,), jnp.float32),
        pltpu.VMEM((N,), jnp.float32),
        pltpu.VMEM_SHARED((NP,), jnp.float32),
        pltpu.SemaphoreType.DMA,
        pltpu.SemaphoreType.DMA,
        pltpu.SemaphoreType.DMA,
        pltpu.SemaphoreType.DMA,
        pltpu.SemaphoreType.DMA,
        pltpu.SemaphoreType.DMA,
        pltpu.SemaphoreType.DMA,
        pltpu.SemaphoreType.DMA,
    ],
    compiler_params=pltpu.CompilerParams(needs_layout_passes=False),
)
def _chain_kernel(sd_hbm, h0_hbm, deg_a, deg_b, s1o, s2o, ws_hbm,
                  wv_hbm, deg0, deg1, sd0, sd1, sd2, sd3, va0, va1, va2,
                  va3, t_a, t_b, w_tile, acc_sp, sm0, sm1, sm2, sm3, si0,
                  si1, si2, si3):
    c = lax.axis_index("c")
    s = lax.axis_index("s")
    off = pl.multiple_of(s * TS, 8)
    sets = ((sd0, va0, sm0, si0), (sd1, va1, sm1, si1),
            (sd2, va2, sm2, si2), (sd3, va3, sm3, si3))

    for h in range(2):
        hsl = pl.ds(pl.multiple_of(off + h * TH, 8), TH)
        pltpu.sync_copy(deg_a.at[hsl], t_a)
        pltpu.sync_copy(deg_b.at[hsl], t_b)

        def ew0(i, _):
            ix = pl.ds(pl.multiple_of(i * 16, 16), 16)
            t_a[ix] = t_a[ix] + t_b[ix] + 1.0
            return 0

        lax.fori_loop(0, TH // 16, ew0, 0)

        @pl.when(c == 0)
        def _():
            pltpu.sync_copy(t_a, deg0.at[hsl])

        @pl.when(c == 1)
        def _():
            pltpu.sync_copy(t_a, deg1.at[hsl])

        def ew1(i, _):
            ix = pl.ds(pl.multiple_of(i * 16, 16), 16)
            t_b[ix] = _rsqrt16(t_a[ix])
            return 0

        lax.fori_loop(0, TH // 16, ew1, 0)

        @pl.when(c == 0)
        def _():
            pltpu.sync_copy(t_b, ws_hbm.at[hsl])
            pltpu.sync_copy(t_b, acc_sp.at[hsl])

        @pl.when(c == 1)
        def _():
            pltpu.sync_copy(h0_hbm.at[hsl], t_a)

            def mul(i, _):
                ix = pl.ds(pl.multiple_of(i * 16, 16), 16)
                t_a[ix] = t_a[ix] * t_b[ix]
                return 0

            lax.fori_loop(0, TH // 16, mul, 0)
            pltpu.sync_copy(t_a, wv_hbm.at[hsl])
            pltpu.sync_copy(t_a, acc_sp.at[hsl])

    plsc.subcore_barrier()

    bbase = s * BLK_PER_TILE

    def edge_pass(w_hbm):
        pltpu.sync_copy(w_hbm.at[pl.ds(0, N)], w_tile)

        def it(i, _):
            b0 = bbase + 4 * i
            idescs = [
                pltpu.async_copy(sd_hbm.at[b0 + k], sdb, semi)
                for k, (sdb, _, _, semi) in enumerate(sets)
            ]
            descs = []
            for k, (sdb, valsb, semx, _) in enumerate(sets):
                idescs[k].wait()
                for j in range(KB):
                    for g in range(8):
                        ix = pl.ds(g * 16, 16)
                        valsb[j, ix] = plsc.load_gather(
                            w_tile, [sdb[j, ix]])
                for j in range(KB):
                    descs.append(
                        pltpu.async_copy(valsb.at[j],
                                         acc_sp.at[sdb.at[KB + j]], semx,
                                         add=True))
            for d in descs:
                d.wait()
            return 0

        lax.fori_loop(0, BLK_PER_TILE // 4, it, 0)

    def ew(out_ref, cont, w_hbm, deg_hbm):
        for h in range(2):
            hsl = pl.ds(pl.multiple_of(off + h * TH, 8), TH)
            pltpu.sync_copy(acc_sp.at[hsl], t_a)
            pltpu.sync_copy(deg_hbm.at[hsl], t_b)

            def body(i, _):
                ix = pl.ds(pl.multiple_of(i * 16, 16), 16)
                a = t_a[ix]
                y = _rsqrt16(t_b[ix])
                if out_ref is not None:
                    t_b[ix] = y * a
                if cont:
                    t_a[ix] = (y * y) * a
                return 0

            lax.fori_loop(0, TH // 16, body, 0)
            if out_ref is not None:
                pltpu.sync_copy(t_b, out_ref.at[hsl])
            if cont:
                pltpu.sync_copy(t_a, w_hbm.at[hsl])
                pltpu.sync_copy(t_a, acc_sp.at[hsl])

    @pl.when(c == 0)
    def _():
        edge_pass(ws_hbm)
        plsc.subcore_barrier()
        ew(s1o, True, ws_hbm, deg0)
        plsc.subcore_barrier()
        edge_pass(ws_hbm)
        plsc.subcore_barrier()
        ew(s2o, False, ws_hbm, deg0)

    @pl.when(c == 1)
    def _():
        edge_pass(wv_hbm)
        plsc.subcore_barrier()
        ew(None, True, wv_hbm, deg1)
        plsc.subcore_barrier()
        edge_pass(wv_hbm)
        plsc.subcore_barrier()
        ew(None, True, wv_hbm, deg1)


@functools.partial(
    pl.kernel,
    out_type=(
        jax.ShapeDtypeStruct((NP,), jnp.float32),
        jax.ShapeDtypeStruct((NP,), jnp.float32),
    ),
    mesh=_SC_MESH,
    scratch_types=[
        pltpu.VMEM((2 * KB, 128), jnp.int32),
        pltpu.VMEM((2 * KB, 128), jnp.int32),
        pltpu.VMEM((KB, 128), jnp.float32),
        pltpu.VMEM((KB, 128), jnp.float32),
        pltpu.VMEM((TH,), jnp.float32),
        pltpu.VMEM((TH,), jnp.float32),
        pltpu.VMEM((N,), jnp.float32),
        pltpu.VMEM_SHARED((NP,), jnp.float32),
        pltpu.SemaphoreType.DMA,
        pltpu.SemaphoreType.DMA,
        pltpu.SemaphoreType.DMA,
        pltpu.SemaphoreType.DMA,
    ],
    compiler_params=pltpu.CompilerParams(needs_layout_passes=False),
)
def _pass3_kernel(sd_hbm, wv_hbm, deg0, deg1, v3a, v3b, sd0, sd1, va0,
                  va1, t_a, t_b, w_tile, acc_sp, sm0, sm1, si0, si1):
    c = lax.axis_index("c")
    s = lax.axis_index("s")
    off = pl.multiple_of(s * TS, 8)
    sets = ((sd0, va0, sm0, si0), (sd1, va1, sm1, si1))

    @pl.when(c == 0)
    def _():
        def fill(i, _):
            ix = pl.ds(pl.multiple_of(i * 16, 16), 16)
            t_a[ix] = jnp.zeros((16,), jnp.float32)
            return 0

        lax.fori_loop(0, TH // 16, fill, 0)
        for h in range(2):
            hsl = pl.ds(pl.multiple_of(off + h * TH, 8), TH)
            pltpu.sync_copy(t_a, acc_sp.at[hsl])

    @pl.when(c == 1)
    def _():
        pltpu.sync_copy(wv_hbm.at[pl.ds(off, TS)],
                        acc_sp.at[pl.ds(off, TS)])

    plsc.subcore_barrier()

    pltpu.sync_copy(wv_hbm.at[pl.ds(0, N)], w_tile)
    bbase = c * (16 * BLK_PER_TILE_HALF) + s * BLK_PER_TILE_HALF

    def it(i, _):
        b0 = bbase + 2 * i
        idescs = [
            pltpu.async_copy(sd_hbm.at[b0 + k], sdb, semi)
            for k, (sdb, _, _, semi) in enumerate(sets)
        ]
        descs = []
        for k, (sdb, valsb, semx, _) in enumerate(sets):
            idescs[k].wait()
            for j in range(KB):
                for g in range(8):
                    ix = pl.ds(g * 16, 16)
                    valsb[j, ix] = plsc.load_gather(w_tile, [sdb[j, ix]])
            for j in range(KB):
                descs.append(
                    pltpu.async_copy(valsb.at[j],
                                     acc_sp.at[sdb.at[KB + j]], semx,
                                     add=True))
        for d in descs:
            d.wait()
        return 0

    lax.fori_loop(0, BLK_PER_TILE_HALF // 2, it, 0)
    plsc.subcore_barrier()

    def final(out_ref, deg_hbm):
        for h in range(2):
            hsl = pl.ds(pl.multiple_of(off + h * TH, 8), TH)
            pltpu.sync_copy(acc_sp.at[hsl], t_a)
            pltpu.sync_copy(deg_hbm.at[hsl], t_b)

            def body(i, _):
                ix = pl.ds(pl.multiple_of(i * 16, 16), 16)
                t_b[ix] = _rsqrt16(t_b[ix]) * t_a[ix]
                return 0

            lax.fori_loop(0, TH // 16, body, 0)
            pltpu.sync_copy(t_b, out_ref.at[hsl])

    @pl.when(c == 0)
    def _():
        final(v3a, deg0)

    @pl.when(c == 1)
    def _():
        final(v3b, deg1)


def _enc_body(x_ref, wi_ref, bi_ref, w2_ref, w3_ref, u1_ref, u2_ref,
              u3_ref, wl1_ref, blr_ref, h0_ref, p4_ref):
    f32 = jnp.float32
    h = jnp.dot(x_ref[...], wi_ref[...], preferred_element_type=f32)
    h0_ref[...] = jax.nn.leaky_relu(h + bi_ref[...])
    t = jnp.dot(u1_ref[...], w2_ref[...], preferred_element_type=f32)
    g = (jnp.dot(t, w3_ref[...], preferred_element_type=f32)
         + jnp.dot(u2_ref[...], w3_ref[...], preferred_element_type=f32)
         + u3_ref[...])
    p4_ref[...] = (jnp.dot(g, wl1_ref[...], preferred_element_type=f32)
                   + blr_ref[...])


def _enc_call(x, wi, bi, w2p, w3p, u1, u2, u3, wl1p, blr):
    return pl.pallas_call(
        _enc_body,
        out_shape=(
            jax.ShapeDtypeStruct((1000, 100), jnp.float32),
            jax.ShapeDtypeStruct((8, 512), jnp.float32),
        ),
    )(x, wi, bi, w2p, w3p, u1, u2, u3, wl1p, blr)


def _dec_body(s4_ref, p4_ref, wl2_ref, bl2_ref, wl3_ref, bl3_ref, out_ref):
    f32 = jnp.float32
    z = jnp.dot(s4_ref[...], p4_ref[...], preferred_element_type=f32)
    z = jax.nn.leaky_relu(z)
    g = jnp.dot(z, wl2_ref[...], preferred_element_type=f32) + bl2_ref[...]
    g = jax.nn.leaky_relu(g)
    o = jnp.dot(g, wl3_ref[...], preferred_element_type=f32) + bl3_ref[...]
    out_ref[...] = jax.nn.leaky_relu(o)


def _dec_call(s4, p4, wl2, bl2, wl3, bl3):
    rows = 4000
    grid = (N // rows,)
    return pl.pallas_call(
        _dec_body,
        grid=grid,
        in_specs=[
            pl.BlockSpec((rows, 8), lambda i: (i, 0)),
            pl.BlockSpec((8, 512), lambda i: (0, 0)),
            pl.BlockSpec((512, 128), lambda i: (0, 0)),
            pl.BlockSpec((1, 128), lambda i: (0, 0)),
            pl.BlockSpec((128, 512), lambda i: (0, 0)),
            pl.BlockSpec((1, 512), lambda i: (0, 0)),
        ],
        out_specs=pl.BlockSpec((rows, 512), lambda i: (i, 0)),
        out_shape=jax.ShapeDtypeStruct((N, 512), jnp.float32),
    )(s4, p4, wl2, bl2, wl3, bl3)


def kernel(x, edge_index, W_inv, b_inv, W1, b1, W2, b2, W3, b3, Wl1, bl1,
           Wl2, bl2, Wl3, bl3):
    f32 = jnp.float32

    pad_dst = N + (jnp.arange(NEP - NE, dtype=jnp.int32) % (NP - N))
    pad_src = jnp.zeros((NEP - NE,), jnp.int32)
    srcp = jnp.concatenate([edge_index[0], pad_src]).reshape(NBLK, KB, 128)
    dstp = jnp.concatenate([edge_index[1], pad_dst]).reshape(NBLK, KB, 128)
    sd = jnp.concatenate([srcp, dstp], axis=1)

    w2p = jnp.zeros((16, 16), f32).at[:9, :3].set(W2)
    w3p = jnp.zeros((16, 16), f32).at[:3, :3].set(W3)
    u1 = (jnp.zeros((8, 16), f32).at[0, :9].set(W1[0])
          .at[1, :9].set(W1[0]).at[2, :9].set(b1))
    u2 = jnp.zeros((8, 16), f32).at[3, :3].set(b2)
    u3 = jnp.zeros((8, 16), f32).at[4, :3].set(b3)
    wl1p = jnp.zeros((16, 512), f32).at[:3].set(Wl1)
    blr = jnp.zeros((8, 512), f32).at[4].set(bl1)

    h0m, p4 = _enc_call(x, W_inv, b_inv.reshape(1, 100), w2p, w3p, u1, u2,
                        u3, wl1p, blr)
    h0p = jnp.concatenate([h0m.reshape(-1), jnp.zeros((NP - N,), f32)])

    deg_a, deg_b = _deg_kernel(sd)
    s1, s2, _, wv, deg0, deg1 = _chain_kernel(sd, h0p, deg_a, deg_b)
    v3a, v3b = _pass3_kernel(sd, wv, deg0, deg1)

    s4 = jnp.stack(
        [v3a[:N], v3b[:N], s2[:N], s1[:N], jnp.ones((N,), f32)], axis=1)
    s4 = jnp.concatenate([s4, jnp.zeros((N, 3), f32)], axis=1)

    out = _dec_call(s4, p4, Wl2, bl2.reshape(1, 128), Wl3,
                    bl3.reshape(1, 512))
    return out, edge_index
